# Initial kernel scaffold; baseline (speedup 1.0000x reference)
import jax
import jax.numpy as jnp
from jax.experimental import pallas as pl


def kernel(x, edge_index_ppi, edge_index, W1, b1, W2, b2, Wc, bc):
    raise NotImplementedError("write your pallas kernel here")


# speedup vs baseline: 17.6358x; 17.6358x over previous
import functools

import jax
import jax.numpy as jnp
from jax import lax
from jax.experimental import pallas as pl
from jax.experimental.pallas import tpu as pltpu
from jax.experimental.pallas import tpu_sc as plsc

N = 10000
E = 320000
EC = 100000
D = 128
H = 128

NC = 2
NS = 16
NW = NC * NS
EPT = E // NW
K = 50
NCHUNK = EPT // K
RPT = N // NS

PPT = 3136
ECP = PPT * NW
PCH = PPT // 16

ZR = 632
ZR_LAST = N - (NS - 1) * ZR

_MESH = plsc.VectorSubcoreMesh(core_axis_name="c", subcore_axis_name="s",
                               num_cores=NC, num_subcores=NS)


def _wid():
    return lax.axis_index("c") * NS + lax.axis_index("s")


def _rowcopy(fn_main, fn_last, s):
    base = pl.multiple_of(s * ZR, 8)

    @pl.when(s < NS - 1)
    def _():
        fn_main(base)

    @pl.when(s == NS - 1)
    def _():
        fn_last()


@functools.partial(
    pl.kernel,
    out_type=jax.ShapeDtypeStruct((NC, N, 16), jnp.float32),
    mesh=_MESH,
    scratch_types=[
        pltpu.VMEM((NCHUNK, K), jnp.int32),
        pltpu.VMEM((K, 16), jnp.float32),
        pltpu.VMEM_SHARED((N, 16), jnp.float32),
    ],
)
def _sc_degree(dst_hbm, ones_hbm, zeros_hbm, parts_hbm, dst_v, ones_v, acc):
    c = lax.axis_index("c")
    s = lax.axis_index("s")
    w = c * NS + s
    pltpu.sync_copy(dst_hbm.at[w], dst_v)
    pltpu.sync_copy(ones_hbm, ones_v)
    _rowcopy(lambda b: pltpu.sync_copy(zeros_hbm.at[pl.ds(b, ZR)],
                                       acc.at[pl.ds(b, ZR)]),
             lambda: pltpu.sync_copy(zeros_hbm.at[pl.ds(N - ZR_LAST, ZR_LAST)],
                                     acc.at[pl.ds(N - ZR_LAST, ZR_LAST)]),
             s)
    plsc.subcore_barrier()

    def body(j, _):
        pltpu.sync_copy(ones_v, acc.at[dst_v.at[j]], add=True)
        return 0

    lax.fori_loop(0, NCHUNK, body, 0)
    plsc.subcore_barrier()
    _rowcopy(lambda b: pltpu.sync_copy(acc.at[pl.ds(b, ZR)],
                                       parts_hbm.at[c, pl.ds(b, ZR)]),
             lambda: pltpu.sync_copy(acc.at[pl.ds(N - ZR_LAST, ZR_LAST)],
                                     parts_hbm.at[c, pl.ds(N - ZR_LAST, ZR_LAST)]),
             s)


HW = H // NC
NCT = 2 * NCHUNK


@functools.partial(
    pl.kernel,
    out_type=jax.ShapeDtypeStruct((NC, N, HW), jnp.float32),
    mesh=_MESH,
    scratch_types=[
        pltpu.VMEM((NCT, K), jnp.int32),
        pltpu.VMEM((NCT, K), jnp.int32),
        pltpu.VMEM((K, HW), jnp.float32),
        pltpu.VMEM((K, HW), jnp.float32),
        pltpu.SemaphoreType.DMA,
        pltpu.SemaphoreType.DMA,
        pltpu.VMEM_SHARED((N, HW), jnp.float32),
    ],
    compiler_params=pltpu.CompilerParams(use_tc_tiling_on_sc=False),
)
def _sc_scatter(g_hbm, src_hbm, dst_hbm, zeros_hbm, parts_hbm,
                src_v, dst_v, rows0, rows1, sem0, sem1, acc):
    c = lax.axis_index("c")
    s = lax.axis_index("s")
    gh = g_hbm.at[c]
    pltpu.sync_copy(src_hbm.at[2 * s], src_v.at[pl.ds(0, NCHUNK)])
    pltpu.sync_copy(src_hbm.at[2 * s + 1], src_v.at[pl.ds(NCHUNK, NCHUNK)])
    pltpu.sync_copy(dst_hbm.at[2 * s], dst_v.at[pl.ds(0, NCHUNK)])
    pltpu.sync_copy(dst_hbm.at[2 * s + 1], dst_v.at[pl.ds(NCHUNK, NCHUNK)])
    _rowcopy(lambda b: pltpu.sync_copy(zeros_hbm.at[pl.ds(b, ZR)],
                                       acc.at[pl.ds(b, ZR)]),
             lambda: pltpu.sync_copy(zeros_hbm.at[pl.ds(N - ZR_LAST, ZR_LAST)],
                                     acc.at[pl.ds(N - ZR_LAST, ZR_LAST)]),
             s)
    plsc.subcore_barrier()

    pltpu.async_copy(gh.at[src_v.at[0]], rows0, sem0)

    def body(j, _):
        @pl.when(j % 2 == 0)
        def _even():
            @pl.when(j + 1 < NCT)
            def _pref():
                pltpu.async_copy(gh.at[src_v.at[j + 1]], rows1, sem1)
            pltpu.make_async_copy(gh.at[src_v.at[j]], rows0, sem0).wait()
            pltpu.sync_copy(rows0, acc.at[dst_v.at[j]], add=True)

        @pl.when(j % 2 == 1)
        def _odd():
            @pl.when(j + 1 < NCT)
            def _pref():
                pltpu.async_copy(gh.at[src_v.at[j + 1]], rows0, sem0)
            pltpu.make_async_copy(gh.at[src_v.at[j]], rows1, sem1).wait()
            pltpu.sync_copy(rows1, acc.at[dst_v.at[j]], add=True)

        return 0

    lax.fori_loop(0, NCT, body, 0)
    plsc.subcore_barrier()
    _rowcopy(lambda b: pltpu.sync_copy(acc.at[pl.ds(b, ZR)],
                                       parts_hbm.at[c, pl.ds(b, ZR)]),
             lambda: pltpu.sync_copy(acc.at[pl.ds(N - ZR_LAST, ZR_LAST)],
                                     parts_hbm.at[c, pl.ds(N - ZR_LAST, ZR_LAST)]),
             s)


@functools.partial(
    pl.kernel,
    out_type=jax.ShapeDtypeStruct((ECP,), jnp.float32),
    mesh=_MESH,
    scratch_types=[
        pltpu.VMEM((N,), jnp.float32),
        pltpu.VMEM((N,), jnp.float32),
        pltpu.VMEM((PPT,), jnp.int32),
        pltpu.VMEM((PPT,), jnp.int32),
        pltpu.VMEM((PPT,), jnp.float32),
    ],
    compiler_params=pltpu.CompilerParams(needs_layout_passes=False),
)
def _sc_pairs(s_hbm, t_hbm, ei0_hbm, ei1_hbm, out_hbm,
              s_v, t_v, i0_v, i1_v, ob_v):
    w = _wid()
    base = pl.multiple_of(w * PPT, 8)
    pltpu.sync_copy(s_hbm, s_v)
    pltpu.sync_copy(t_hbm, t_v)
    pltpu.sync_copy(ei0_hbm.at[pl.ds(base, PPT)], i0_v)
    pltpu.sync_copy(ei1_hbm.at[pl.ds(base, PPT)], i1_v)

    def body(j, _):
        sl = pl.ds(j * 16, 16)
        v0 = plsc.load_gather(s_v, [i0_v[sl]])
        v1 = plsc.load_gather(t_v, [i1_v[sl]])
        z = v0 + v1
        ob_v[sl] = 1.0 / (1.0 + jnp.exp(-z))
        return 0

    lax.fori_loop(0, PCH, body, 0)
    pltpu.sync_copy(ob_v, out_hbm.at[pl.ds(base, PPT)])


BN = 2000


def _dinv_body(parts_ref, dinv_ref):
    deg = parts_ref[0, :, 0:1] + parts_ref[1, :, 0:1] + 1.0
    dinv_ref[...] = lax.rsqrt(deg)


def _tc_dinv(parts):
    return pl.pallas_call(
        _dinv_body,
        grid=(N // BN,),
        in_specs=[pl.BlockSpec((NC, BN, 16), lambda i: (0, i, 0))],
        out_specs=pl.BlockSpec((BN, 1), lambda i: (i, 0)),
        out_shape=jax.ShapeDtypeStruct((N, 1), jnp.float32),
    )(parts)


def _split_store(out_ref, g):
    out_ref[0] = g[:, :HW]
    out_ref[1] = g[:, HW:]


def _scale_mm_body(x_ref, w_ref, dinv_ref, g_ref):
    xw = jnp.dot(x_ref[...], w_ref[...], preferred_element_type=jnp.float32)
    _split_store(g_ref, dinv_ref[...] * xw)


def _tc_scale_mm(x, W, dinv):
    return pl.pallas_call(
        _scale_mm_body,
        grid=(N // BN,),
        in_specs=[
            pl.BlockSpec((BN, D), lambda i: (i, 0)),
            pl.BlockSpec((D, H), lambda i: (0, 0)),
            pl.BlockSpec((BN, 1), lambda i: (i, 0)),
        ],
        out_specs=pl.BlockSpec((NC, BN, HW), lambda i: (0, i, 0)),
        out_shape=jax.ShapeDtypeStruct((NC, N, HW), jnp.float32),
    )(x, W, dinv)


def _layer_body(parts_ref, g_ref, dinv_ref, b_ref, w_ref, out_ref):
    tot = jnp.concatenate([parts_ref[0] + g_ref[0], parts_ref[1] + g_ref[1]],
                          axis=1)
    h = jnp.maximum(dinv_ref[...] * tot + b_ref[...], 0.0)
    hw = jnp.dot(h, w_ref[...], preferred_element_type=jnp.float32)
    _split_store(out_ref, dinv_ref[...] * hw)


def _tc_layer(parts, g, dinv, b, W):
    return pl.pallas_call(
        _layer_body,
        grid=(N // BN,),
        in_specs=[
            pl.BlockSpec((NC, BN, HW), lambda i: (0, i, 0)),
            pl.BlockSpec((NC, BN, HW), lambda i: (0, i, 0)),
            pl.BlockSpec((BN, 1), lambda i: (i, 0)),
            pl.BlockSpec((1, H), lambda i: (0, 0)),
            pl.BlockSpec((H, H), lambda i: (0, 0)),
        ],
        out_specs=pl.BlockSpec((NC, BN, HW), lambda i: (0, i, 0)),
        out_shape=jax.ShapeDtypeStruct((NC, N, HW), jnp.float32),
    )(parts, g, dinv, b, W)


def _final_body(parts_ref, g_ref, dinv_ref, b_ref, wc0_ref, wc1_ref, bc_ref,
                h_ref, s_ref, t_ref):
    tot = jnp.concatenate([parts_ref[0] + g_ref[0], parts_ref[1] + g_ref[1]],
                          axis=1)
    h = jnp.maximum(dinv_ref[...] * tot + b_ref[...], 0.0)
    h_ref[...] = h
    s_ref[...] = jnp.sum(h * wc0_ref[...], axis=1, keepdims=True) + bc_ref[0]
    t_ref[...] = jnp.sum(h * wc1_ref[...], axis=1, keepdims=True)


def _tc_final(parts, g, dinv, b, wc0, wc1, bc):
    return pl.pallas_call(
        _final_body,
        grid=(N // BN,),
        in_specs=[
            pl.BlockSpec((NC, BN, HW), lambda i: (0, i, 0)),
            pl.BlockSpec((NC, BN, HW), lambda i: (0, i, 0)),
            pl.BlockSpec((BN, 1), lambda i: (i, 0)),
            pl.BlockSpec((1, H), lambda i: (0, 0)),
            pl.BlockSpec((1, H), lambda i: (0, 0)),
            pl.BlockSpec((1, H), lambda i: (0, 0)),
            pl.BlockSpec(memory_space=pltpu.SMEM),
        ],
        out_specs=[
            pl.BlockSpec((BN, H), lambda i: (i, 0)),
            pl.BlockSpec((BN, 1), lambda i: (i, 0)),
            pl.BlockSpec((BN, 1), lambda i: (i, 0)),
        ],
        out_shape=[
            jax.ShapeDtypeStruct((N, H), jnp.float32),
            jax.ShapeDtypeStruct((N, 1), jnp.float32),
            jax.ShapeDtypeStruct((N, 1), jnp.float32),
        ],
    )(parts, g, dinv, b, wc0, wc1, bc)


def kernel(x, edge_index_ppi, edge_index, W1, b1, W2, b2, Wc, bc):
    src = edge_index_ppi[0].reshape(NW, NCHUNK, K)
    dst = edge_index_ppi[1].reshape(NW, NCHUNK, K)

    ones16 = jnp.ones((K, 16), jnp.float32)
    zeros16 = jnp.zeros((N, 16), jnp.float32)
    zerosHW = jnp.zeros((N, HW), jnp.float32)

    deg_parts = _sc_degree(dst, ones16, zeros16)
    dinv = _tc_dinv(deg_parts)

    g1 = _tc_scale_mm(x, W1, dinv)
    p1 = _sc_scatter(g1, src, dst, zerosHW)
    g2 = _tc_layer(p1, g1, dinv, b1.reshape(1, H), W2)
    p2 = _sc_scatter(g2, src, dst, zerosHW)

    wc0 = Wc[:H, 0].reshape(1, H)
    wc1 = Wc[H:, 0].reshape(1, H)
    h2, s_col, t_col = _tc_final(p2, g2, dinv, b2.reshape(1, H), wc0, wc1, bc)

    pad = ECP - EC
    ei0 = jnp.pad(edge_index[0], (0, pad))
    ei1 = jnp.pad(edge_index[1], (0, pad))
    probs = _sc_pairs(s_col.reshape(N), t_col.reshape(N), ei0, ei1)
    return (h2, probs[:EC].reshape(EC, 1))


# speedup vs baseline: 17.8246x; 1.0107x over previous
import functools

import jax
import jax.numpy as jnp
from jax import lax
from jax.experimental import pallas as pl
from jax.experimental.pallas import tpu as pltpu
from jax.experimental.pallas import tpu_sc as plsc

N = 10000
E = 320000
EC = 100000
D = 128
H = 128

NC = 2
NS = 16
NW = NC * NS
EPT = E // NW
K = 50
NCHUNK = EPT // K
RPT = N // NS

PPT = 3136
ECP = PPT * NW
PCH = PPT // 16

ZR = 632
ZR_LAST = N - (NS - 1) * ZR

_MESH = plsc.VectorSubcoreMesh(core_axis_name="c", subcore_axis_name="s",
                               num_cores=NC, num_subcores=NS)


def _wid():
    return lax.axis_index("c") * NS + lax.axis_index("s")


def _rowcopy(fn_main, fn_last, s):
    base = pl.multiple_of(s * ZR, 8)

    @pl.when(s < NS - 1)
    def _():
        fn_main(base)

    @pl.when(s == NS - 1)
    def _():
        fn_last()


@functools.partial(
    pl.kernel,
    out_type=jax.ShapeDtypeStruct((NC, N, 16), jnp.float32),
    mesh=_MESH,
    scratch_types=[
        pltpu.VMEM((NCHUNK, K), jnp.int32),
        pltpu.VMEM((K, 16), jnp.float32),
        pltpu.VMEM_SHARED((N, 16), jnp.float32),
    ],
    compiler_params=pltpu.CompilerParams(use_tc_tiling_on_sc=False),
)
def _sc_degree(dst_hbm, ones_hbm, zeros_hbm, parts_hbm, dst_v, ones_v, acc):
    c = lax.axis_index("c")
    s = lax.axis_index("s")
    w = c * NS + s
    pltpu.sync_copy(dst_hbm.at[w], dst_v)
    pltpu.sync_copy(ones_hbm, ones_v)
    _rowcopy(lambda b: pltpu.sync_copy(zeros_hbm.at[pl.ds(b, ZR)],
                                       acc.at[pl.ds(b, ZR)]),
             lambda: pltpu.sync_copy(zeros_hbm.at[pl.ds(N - ZR_LAST, ZR_LAST)],
                                     acc.at[pl.ds(N - ZR_LAST, ZR_LAST)]),
             s)
    plsc.subcore_barrier()

    def body(j, _):
        pltpu.sync_copy(ones_v, acc.at[dst_v.at[j]], add=True)
        return 0

    lax.fori_loop(0, NCHUNK, body, 0)
    plsc.subcore_barrier()
    _rowcopy(lambda b: pltpu.sync_copy(acc.at[pl.ds(b, ZR)],
                                       parts_hbm.at[c, pl.ds(b, ZR)]),
             lambda: pltpu.sync_copy(acc.at[pl.ds(N - ZR_LAST, ZR_LAST)],
                                     parts_hbm.at[c, pl.ds(N - ZR_LAST, ZR_LAST)]),
             s)


HW = H // NC
NCT = 2 * NCHUNK


@functools.partial(
    pl.kernel,
    out_type=jax.ShapeDtypeStruct((NC, N, HW), jnp.float32),
    mesh=_MESH,
    scratch_types=[
        pltpu.VMEM((NCT, K), jnp.int32),
        pltpu.VMEM((NCT, K), jnp.int32),
        pltpu.VMEM((K, HW), jnp.float32),
        pltpu.VMEM((K, HW), jnp.float32),
        pltpu.SemaphoreType.DMA,
        pltpu.SemaphoreType.DMA,
        pltpu.VMEM_SHARED((N, HW), jnp.float32),
    ],
    compiler_params=pltpu.CompilerParams(use_tc_tiling_on_sc=False),
)
def _sc_scatter(g_hbm, src_hbm, dst_hbm, zeros_hbm, parts_hbm,
                src_v, dst_v, rows0, rows1, sem0, sem1, acc):
    c = lax.axis_index("c")
    s = lax.axis_index("s")
    gh = g_hbm.at[c]
    pltpu.sync_copy(src_hbm.at[2 * s], src_v.at[pl.ds(0, NCHUNK)])
    pltpu.sync_copy(src_hbm.at[2 * s + 1], src_v.at[pl.ds(NCHUNK, NCHUNK)])
    pltpu.sync_copy(dst_hbm.at[2 * s], dst_v.at[pl.ds(0, NCHUNK)])
    pltpu.sync_copy(dst_hbm.at[2 * s + 1], dst_v.at[pl.ds(NCHUNK, NCHUNK)])
    _rowcopy(lambda b: pltpu.sync_copy(zeros_hbm.at[pl.ds(b, ZR)],
                                       acc.at[pl.ds(b, ZR)]),
             lambda: pltpu.sync_copy(zeros_hbm.at[pl.ds(N - ZR_LAST, ZR_LAST)],
                                     acc.at[pl.ds(N - ZR_LAST, ZR_LAST)]),
             s)
    plsc.subcore_barrier()

    pltpu.async_copy(gh.at[src_v.at[0]], rows0, sem0)

    def body(j, _):
        @pl.when(j % 2 == 0)
        def _even():
            @pl.when(j + 1 < NCT)
            def _pref():
                pltpu.async_copy(gh.at[src_v.at[j + 1]], rows1, sem1)
            pltpu.make_async_copy(gh.at[src_v.at[j]], rows0, sem0).wait()
            pltpu.sync_copy(rows0, acc.at[dst_v.at[j]], add=True)

        @pl.when(j % 2 == 1)
        def _odd():
            @pl.when(j + 1 < NCT)
            def _pref():
                pltpu.async_copy(gh.at[src_v.at[j + 1]], rows0, sem0)
            pltpu.make_async_copy(gh.at[src_v.at[j]], rows1, sem1).wait()
            pltpu.sync_copy(rows1, acc.at[dst_v.at[j]], add=True)

        return 0

    lax.fori_loop(0, NCT, body, 0)
    plsc.subcore_barrier()
    _rowcopy(lambda b: pltpu.sync_copy(acc.at[pl.ds(b, ZR)],
                                       parts_hbm.at[c, pl.ds(b, ZR)]),
             lambda: pltpu.sync_copy(acc.at[pl.ds(N - ZR_LAST, ZR_LAST)],
                                     parts_hbm.at[c, pl.ds(N - ZR_LAST, ZR_LAST)]),
             s)


@functools.partial(
    pl.kernel,
    out_type=jax.ShapeDtypeStruct((ECP,), jnp.float32),
    mesh=_MESH,
    scratch_types=[
        pltpu.VMEM((N,), jnp.float32),
        pltpu.VMEM((N,), jnp.float32),
        pltpu.VMEM((PPT,), jnp.int32),
        pltpu.VMEM((PPT,), jnp.int32),
        pltpu.VMEM((PPT,), jnp.float32),
    ],
    compiler_params=pltpu.CompilerParams(needs_layout_passes=False),
)
def _sc_pairs(s_hbm, t_hbm, ei0_hbm, ei1_hbm, out_hbm,
              s_v, t_v, i0_v, i1_v, ob_v):
    w = _wid()
    base = pl.multiple_of(w * PPT, 8)
    pltpu.sync_copy(s_hbm, s_v)
    pltpu.sync_copy(t_hbm, t_v)
    pltpu.sync_copy(ei0_hbm.at[pl.ds(base, PPT)], i0_v)
    pltpu.sync_copy(ei1_hbm.at[pl.ds(base, PPT)], i1_v)

    def body(j, _):
        sl = pl.ds(j * 16, 16)
        v0 = plsc.load_gather(s_v, [i0_v[sl]])
        v1 = plsc.load_gather(t_v, [i1_v[sl]])
        z = v0 + v1
        ob_v[sl] = 1.0 / (1.0 + jnp.exp(-z))
        return 0

    lax.fori_loop(0, PCH, body, 0)
    pltpu.sync_copy(ob_v, out_hbm.at[pl.ds(base, PPT)])


BN = 2000


def _dinv_of(deg_ref):
    deg = deg_ref[0, :, 0:1] + deg_ref[1, :, 0:1] + 1.0
    return lax.rsqrt(deg)


_DEG_SPEC = pl.BlockSpec((NC, BN, 16), lambda i: (0, i, 0))


def _split_store(out_ref, g):
    out_ref[0] = g[:, :HW]
    out_ref[1] = g[:, HW:]


def _scale_mm_body(x_ref, w_ref, deg_ref, g_ref):
    xw = jnp.dot(x_ref[...], w_ref[...], preferred_element_type=jnp.float32)
    _split_store(g_ref, _dinv_of(deg_ref) * xw)


def _tc_scale_mm(x, W, deg_parts):
    return pl.pallas_call(
        _scale_mm_body,
        grid=(N // BN,),
        in_specs=[
            pl.BlockSpec((BN, D), lambda i: (i, 0)),
            pl.BlockSpec((D, H), lambda i: (0, 0)),
            _DEG_SPEC,
        ],
        out_specs=pl.BlockSpec((NC, BN, HW), lambda i: (0, i, 0)),
        out_shape=jax.ShapeDtypeStruct((NC, N, HW), jnp.float32),
    )(x, W, deg_parts)


def _layer_body(parts_ref, g_ref, deg_ref, b_ref, w_ref, out_ref):
    dinv = _dinv_of(deg_ref)
    tot = jnp.concatenate([parts_ref[0] + g_ref[0], parts_ref[1] + g_ref[1]],
                          axis=1)
    h = jnp.maximum(dinv * tot + b_ref[...], 0.0)
    hw = jnp.dot(h, w_ref[...], preferred_element_type=jnp.float32)
    _split_store(out_ref, dinv * hw)


def _tc_layer(parts, g, deg_parts, b, W):
    return pl.pallas_call(
        _layer_body,
        grid=(N // BN,),
        in_specs=[
            pl.BlockSpec((NC, BN, HW), lambda i: (0, i, 0)),
            pl.BlockSpec((NC, BN, HW), lambda i: (0, i, 0)),
            _DEG_SPEC,
            pl.BlockSpec((1, H), lambda i: (0, 0)),
            pl.BlockSpec((H, H), lambda i: (0, 0)),
        ],
        out_specs=pl.BlockSpec((NC, BN, HW), lambda i: (0, i, 0)),
        out_shape=jax.ShapeDtypeStruct((NC, N, HW), jnp.float32),
    )(parts, g, deg_parts, b, W)


def _final_body(parts_ref, g_ref, deg_ref, b_ref, wc0_ref, wc1_ref, bc_ref,
                h_ref, s_ref, t_ref):
    dinv = _dinv_of(deg_ref)
    tot = jnp.concatenate([parts_ref[0] + g_ref[0], parts_ref[1] + g_ref[1]],
                          axis=1)
    h = jnp.maximum(dinv * tot + b_ref[...], 0.0)
    h_ref[...] = h
    s_ref[...] = jnp.sum(h * wc0_ref[...], axis=1, keepdims=True) + bc_ref[0]
    t_ref[...] = jnp.sum(h * wc1_ref[...], axis=1, keepdims=True)


def _tc_final(parts, g, deg_parts, b, wc0, wc1, bc):
    return pl.pallas_call(
        _final_body,
        grid=(N // BN,),
        in_specs=[
            pl.BlockSpec((NC, BN, HW), lambda i: (0, i, 0)),
            pl.BlockSpec((NC, BN, HW), lambda i: (0, i, 0)),
            _DEG_SPEC,
            pl.BlockSpec((1, H), lambda i: (0, 0)),
            pl.BlockSpec((1, H), lambda i: (0, 0)),
            pl.BlockSpec((1, H), lambda i: (0, 0)),
            pl.BlockSpec(memory_space=pltpu.SMEM),
        ],
        out_specs=[
            pl.BlockSpec((BN, H), lambda i: (i, 0)),
            pl.BlockSpec((BN, 1), lambda i: (i, 0)),
            pl.BlockSpec((BN, 1), lambda i: (i, 0)),
        ],
        out_shape=[
            jax.ShapeDtypeStruct((N, H), jnp.float32),
            jax.ShapeDtypeStruct((N, 1), jnp.float32),
            jax.ShapeDtypeStruct((N, 1), jnp.float32),
        ],
    )(parts, g, deg_parts, b, wc0, wc1, bc)


def kernel(x, edge_index_ppi, edge_index, W1, b1, W2, b2, Wc, bc):
    src = edge_index_ppi[0].reshape(NW, NCHUNK, K)
    dst = edge_index_ppi[1].reshape(NW, NCHUNK, K)

    ones16 = jnp.ones((K, 16), jnp.float32)
    zeros16 = jnp.zeros((N, 16), jnp.float32)
    zerosHW = jnp.zeros((N, HW), jnp.float32)

    deg_parts = _sc_degree(dst, ones16, zeros16)

    g1 = _tc_scale_mm(x, W1, deg_parts)
    p1 = _sc_scatter(g1, src, dst, zerosHW)
    g2 = _tc_layer(p1, g1, deg_parts, b1.reshape(1, H), W2)
    p2 = _sc_scatter(g2, src, dst, zerosHW)

    wc0 = Wc[:H, 0].reshape(1, H)
    wc1 = Wc[H:, 0].reshape(1, H)
    h2, s_col, t_col = _tc_final(p2, g2, deg_parts, b2.reshape(1, H), wc0,
                                 wc1, bc)

    pad = ECP - EC
    ei0 = jnp.pad(edge_index[0], (0, pad))
    ei1 = jnp.pad(edge_index[1], (0, pad))
    probs = _sc_pairs(s_col.reshape(N), t_col.reshape(N), ei0, ei1)
    return (h2, probs[:EC].reshape(EC, 1))


# speedup vs baseline: 23.5450x; 1.3209x over previous
import functools

import jax
import jax.numpy as jnp
from jax import lax
from jax.experimental import pallas as pl
from jax.experimental.pallas import tpu as pltpu
from jax.experimental.pallas import tpu_sc as plsc

N = 10000
E = 320000
EC = 100000
D = 128
H = 128

NC = 2
NS = 16
NW = NC * NS
EPT = E // NW
K = 100
NCHUNK = EPT // K
RPT = N // NS

PPT = 3136
ECP = PPT * NW
PCH = PPT // 16

ZR = 632
ZR_LAST = N - (NS - 1) * ZR

_MESH = plsc.VectorSubcoreMesh(core_axis_name="c", subcore_axis_name="s",
                               num_cores=NC, num_subcores=NS)


def _wid():
    return lax.axis_index("c") * NS + lax.axis_index("s")


def _rowcopy(fn_main, fn_last, s):
    base = pl.multiple_of(s * ZR, 8)

    @pl.when(s < NS - 1)
    def _():
        fn_main(base)

    @pl.when(s == NS - 1)
    def _():
        fn_last()


@functools.partial(
    pl.kernel,
    out_type=jax.ShapeDtypeStruct((NC, N, 16), jnp.float32),
    mesh=_MESH,
    scratch_types=[
        pltpu.VMEM((NCHUNK, K), jnp.int32),
        pltpu.VMEM((K, 16), jnp.float32),
        pltpu.VMEM_SHARED((N, 16), jnp.float32),
    ],
    compiler_params=pltpu.CompilerParams(use_tc_tiling_on_sc=False),
)
def _sc_degree(dst_hbm, ones_hbm, zeros_hbm, parts_hbm, dst_v, ones_v, acc):
    c = lax.axis_index("c")
    s = lax.axis_index("s")
    w = c * NS + s
    pltpu.sync_copy(dst_hbm.at[w], dst_v)
    pltpu.sync_copy(ones_hbm, ones_v)
    _rowcopy(lambda b: pltpu.sync_copy(zeros_hbm.at[pl.ds(b, ZR)],
                                       acc.at[pl.ds(b, ZR)]),
             lambda: pltpu.sync_copy(zeros_hbm.at[pl.ds(N - ZR_LAST, ZR_LAST)],
                                     acc.at[pl.ds(N - ZR_LAST, ZR_LAST)]),
             s)
    plsc.subcore_barrier()

    def body(j, _):
        pltpu.sync_copy(ones_v, acc.at[dst_v.at[j]], add=True)
        return 0

    lax.fori_loop(0, NCHUNK, body, 0)
    plsc.subcore_barrier()
    _rowcopy(lambda b: pltpu.sync_copy(acc.at[pl.ds(b, ZR)],
                                       parts_hbm.at[c, pl.ds(b, ZR)]),
             lambda: pltpu.sync_copy(acc.at[pl.ds(N - ZR_LAST, ZR_LAST)],
                                     parts_hbm.at[c, pl.ds(N - ZR_LAST, ZR_LAST)]),
             s)


HW = H // NC
NCT = 2 * NCHUNK


@functools.partial(
    pl.kernel,
    out_type=jax.ShapeDtypeStruct((NC, N, HW), jnp.float32),
    mesh=_MESH,
    scratch_types=[
        pltpu.VMEM((NCT, K), jnp.int32),
        pltpu.VMEM((NCT, K), jnp.int32),
        pltpu.VMEM((K, HW), jnp.float32),
        pltpu.VMEM((K, HW), jnp.float32),
        pltpu.SemaphoreType.DMA,
        pltpu.SemaphoreType.DMA,
        pltpu.VMEM_SHARED((N, HW), jnp.float32),
    ],
    compiler_params=pltpu.CompilerParams(use_tc_tiling_on_sc=False),
)
def _sc_scatter(g_hbm, src_hbm, dst_hbm, zeros_hbm, parts_hbm,
                src_v, dst_v, rows0, rows1, sem0, sem1, acc):
    c = lax.axis_index("c")
    s = lax.axis_index("s")
    gh = g_hbm.at[c]
    pltpu.sync_copy(src_hbm.at[2 * s], src_v.at[pl.ds(0, NCHUNK)])
    pltpu.sync_copy(src_hbm.at[2 * s + 1], src_v.at[pl.ds(NCHUNK, NCHUNK)])
    pltpu.sync_copy(dst_hbm.at[2 * s], dst_v.at[pl.ds(0, NCHUNK)])
    pltpu.sync_copy(dst_hbm.at[2 * s + 1], dst_v.at[pl.ds(NCHUNK, NCHUNK)])
    _rowcopy(lambda b: pltpu.sync_copy(zeros_hbm.at[pl.ds(b, ZR)],
                                       acc.at[pl.ds(b, ZR)]),
             lambda: pltpu.sync_copy(zeros_hbm.at[pl.ds(N - ZR_LAST, ZR_LAST)],
                                     acc.at[pl.ds(N - ZR_LAST, ZR_LAST)]),
             s)
    plsc.subcore_barrier()

    pltpu.async_copy(gh.at[src_v.at[0]], rows0, sem0)

    def body(j, _):
        @pl.when(j % 2 == 0)
        def _even():
            @pl.when(j + 1 < NCT)
            def _pref():
                pltpu.async_copy(gh.at[src_v.at[j + 1]], rows1, sem1)
            pltpu.make_async_copy(gh.at[src_v.at[j]], rows0, sem0).wait()
            pltpu.sync_copy(rows0, acc.at[dst_v.at[j]], add=True)

        @pl.when(j % 2 == 1)
        def _odd():
            @pl.when(j + 1 < NCT)
            def _pref():
                pltpu.async_copy(gh.at[src_v.at[j + 1]], rows0, sem0)
            pltpu.make_async_copy(gh.at[src_v.at[j]], rows1, sem1).wait()
            pltpu.sync_copy(rows1, acc.at[dst_v.at[j]], add=True)

        return 0

    lax.fori_loop(0, NCT, body, 0)
    plsc.subcore_barrier()
    _rowcopy(lambda b: pltpu.sync_copy(acc.at[pl.ds(b, ZR)],
                                       parts_hbm.at[c, pl.ds(b, ZR)]),
             lambda: pltpu.sync_copy(acc.at[pl.ds(N - ZR_LAST, ZR_LAST)],
                                     parts_hbm.at[c, pl.ds(N - ZR_LAST, ZR_LAST)]),
             s)


@functools.partial(
    pl.kernel,
    out_type=jax.ShapeDtypeStruct((ECP,), jnp.float32),
    mesh=_MESH,
    scratch_types=[
        pltpu.VMEM((N,), jnp.float32),
        pltpu.VMEM((N,), jnp.float32),
        pltpu.VMEM((PPT,), jnp.int32),
        pltpu.VMEM((PPT,), jnp.int32),
        pltpu.VMEM((PPT,), jnp.float32),
    ],
    compiler_params=pltpu.CompilerParams(needs_layout_passes=False),
)
def _sc_pairs(s_hbm, t_hbm, ei0_hbm, ei1_hbm, out_hbm,
              s_v, t_v, i0_v, i1_v, ob_v):
    w = _wid()
    base = pl.multiple_of(w * PPT, 8)
    pltpu.sync_copy(s_hbm, s_v)
    pltpu.sync_copy(t_hbm, t_v)
    pltpu.sync_copy(ei0_hbm.at[pl.ds(base, PPT)], i0_v)
    pltpu.sync_copy(ei1_hbm.at[pl.ds(base, PPT)], i1_v)

    def body(j, _):
        sl = pl.ds(j * 16, 16)
        v0 = plsc.load_gather(s_v, [i0_v[sl]])
        v1 = plsc.load_gather(t_v, [i1_v[sl]])
        z = v0 + v1
        ob_v[sl] = 1.0 / (1.0 + jnp.exp(-z))
        return 0

    lax.fori_loop(0, PCH, body, 0)
    pltpu.sync_copy(ob_v, out_hbm.at[pl.ds(base, PPT)])


BN = 2000


def _dinv_of(deg_ref):
    deg = deg_ref[0, :, 0:1] + deg_ref[1, :, 0:1] + 1.0
    return lax.rsqrt(deg)


_DEG_SPEC = pl.BlockSpec((NC, BN, 16), lambda i: (0, i, 0))


def _split_store(out_ref, g):
    out_ref[0] = g[:, :HW]
    out_ref[1] = g[:, HW:]


def _scale_mm_body(x_ref, w_ref, deg_ref, g_ref):
    xw = jnp.dot(x_ref[...], w_ref[...], preferred_element_type=jnp.float32)
    _split_store(g_ref, _dinv_of(deg_ref) * xw)


def _tc_scale_mm(x, W, deg_parts):
    return pl.pallas_call(
        _scale_mm_body,
        grid=(N // BN,),
        in_specs=[
            pl.BlockSpec((BN, D), lambda i: (i, 0)),
            pl.BlockSpec((D, H), lambda i: (0, 0)),
            _DEG_SPEC,
        ],
        out_specs=pl.BlockSpec((NC, BN, HW), lambda i: (0, i, 0)),
        out_shape=jax.ShapeDtypeStruct((NC, N, HW), jnp.float32),
    )(x, W, deg_parts)


def _layer_body(parts_ref, g_ref, deg_ref, b_ref, w_ref, out_ref):
    dinv = _dinv_of(deg_ref)
    tot = jnp.concatenate([parts_ref[0] + g_ref[0], parts_ref[1] + g_ref[1]],
                          axis=1)
    h = jnp.maximum(dinv * tot + b_ref[...], 0.0)
    hw = jnp.dot(h, w_ref[...], preferred_element_type=jnp.float32)
    _split_store(out_ref, dinv * hw)


def _tc_layer(parts, g, deg_parts, b, W):
    return pl.pallas_call(
        _layer_body,
        grid=(N // BN,),
        in_specs=[
            pl.BlockSpec((NC, BN, HW), lambda i: (0, i, 0)),
            pl.BlockSpec((NC, BN, HW), lambda i: (0, i, 0)),
            _DEG_SPEC,
            pl.BlockSpec((1, H), lambda i: (0, 0)),
            pl.BlockSpec((H, H), lambda i: (0, 0)),
        ],
        out_specs=pl.BlockSpec((NC, BN, HW), lambda i: (0, i, 0)),
        out_shape=jax.ShapeDtypeStruct((NC, N, HW), jnp.float32),
    )(parts, g, deg_parts, b, W)


def _final_body(parts_ref, g_ref, deg_ref, b_ref, wc0_ref, wc1_ref, bc_ref,
                h_ref, s_ref, t_ref):
    dinv = _dinv_of(deg_ref)
    tot = jnp.concatenate([parts_ref[0] + g_ref[0], parts_ref[1] + g_ref[1]],
                          axis=1)
    h = jnp.maximum(dinv * tot + b_ref[...], 0.0)
    h_ref[...] = h
    s_ref[...] = jnp.sum(h * wc0_ref[...], axis=1, keepdims=True) + bc_ref[0]
    t_ref[...] = jnp.sum(h * wc1_ref[...], axis=1, keepdims=True)


def _tc_final(parts, g, deg_parts, b, wc0, wc1, bc):
    return pl.pallas_call(
        _final_body,
        grid=(N // BN,),
        in_specs=[
            pl.BlockSpec((NC, BN, HW), lambda i: (0, i, 0)),
            pl.BlockSpec((NC, BN, HW), lambda i: (0, i, 0)),
            _DEG_SPEC,
            pl.BlockSpec((1, H), lambda i: (0, 0)),
            pl.BlockSpec((1, H), lambda i: (0, 0)),
            pl.BlockSpec((1, H), lambda i: (0, 0)),
            pl.BlockSpec(memory_space=pltpu.SMEM),
        ],
        out_specs=[
            pl.BlockSpec((BN, H), lambda i: (i, 0)),
            pl.BlockSpec((BN, 1), lambda i: (i, 0)),
            pl.BlockSpec((BN, 1), lambda i: (i, 0)),
        ],
        out_shape=[
            jax.ShapeDtypeStruct((N, H), jnp.float32),
            jax.ShapeDtypeStruct((N, 1), jnp.float32),
            jax.ShapeDtypeStruct((N, 1), jnp.float32),
        ],
    )(parts, g, deg_parts, b, wc0, wc1, bc)


def kernel(x, edge_index_ppi, edge_index, W1, b1, W2, b2, Wc, bc):
    src = edge_index_ppi[0].reshape(NW, NCHUNK, K)
    dst = edge_index_ppi[1].reshape(NW, NCHUNK, K)

    ones16 = jnp.ones((K, 16), jnp.float32)
    zeros16 = jnp.zeros((N, 16), jnp.float32)
    zerosHW = jnp.zeros((N, HW), jnp.float32)

    deg_parts = _sc_degree(dst, ones16, zeros16)

    g1 = _tc_scale_mm(x, W1, deg_parts)
    p1 = _sc_scatter(g1, src, dst, zerosHW)
    g2 = _tc_layer(p1, g1, deg_parts, b1.reshape(1, H), W2)
    p2 = _sc_scatter(g2, src, dst, zerosHW)

    wc0 = Wc[:H, 0].reshape(1, H)
    wc1 = Wc[H:, 0].reshape(1, H)
    h2, s_col, t_col = _tc_final(p2, g2, deg_parts, b2.reshape(1, H), wc0,
                                 wc1, bc)

    pad = ECP - EC
    ei0 = jnp.pad(edge_index[0], (0, pad))
    ei1 = jnp.pad(edge_index[1], (0, pad))
    probs = _sc_pairs(s_col.reshape(N), t_col.reshape(N), ei0, ei1)
    return (h2, probs[:EC].reshape(EC, 1))


# speedup vs baseline: 25.0450x; 1.0637x over previous
import functools

import jax
import jax.numpy as jnp
from jax import lax
from jax.experimental import pallas as pl
from jax.experimental.pallas import tpu as pltpu
from jax.experimental.pallas import tpu_sc as plsc

N = 10000
E = 320000
EC = 100000
D = 128
H = 128

NC = 2
NS = 16
NW = NC * NS
EPT = E // NW
K = 125
NCHUNK = EPT // K
RPT = N // NS

PPT = 3136
ECP = PPT * NW
PCH = PPT // 16

ZR = 632
ZR_LAST = N - (NS - 1) * ZR

_MESH = plsc.VectorSubcoreMesh(core_axis_name="c", subcore_axis_name="s",
                               num_cores=NC, num_subcores=NS)


def _wid():
    return lax.axis_index("c") * NS + lax.axis_index("s")


def _rowcopy(fn_main, fn_last, s):
    base = pl.multiple_of(s * ZR, 8)

    @pl.when(s < NS - 1)
    def _():
        fn_main(base)

    @pl.when(s == NS - 1)
    def _():
        fn_last()


@functools.partial(
    pl.kernel,
    out_type=jax.ShapeDtypeStruct((NC, N, 16), jnp.float32),
    mesh=_MESH,
    scratch_types=[
        pltpu.VMEM((NCHUNK, K), jnp.int32),
        pltpu.VMEM((K, 16), jnp.float32),
        pltpu.VMEM_SHARED((N, 16), jnp.float32),
    ],
    compiler_params=pltpu.CompilerParams(use_tc_tiling_on_sc=False),
)
def _sc_degree(dst_hbm, ones_hbm, zeros_hbm, parts_hbm, dst_v, ones_v, acc):
    c = lax.axis_index("c")
    s = lax.axis_index("s")
    w = c * NS + s
    pltpu.sync_copy(dst_hbm.at[w], dst_v)
    pltpu.sync_copy(ones_hbm, ones_v)
    _rowcopy(lambda b: pltpu.sync_copy(zeros_hbm.at[pl.ds(b, ZR)],
                                       acc.at[pl.ds(b, ZR)]),
             lambda: pltpu.sync_copy(zeros_hbm.at[pl.ds(N - ZR_LAST, ZR_LAST)],
                                     acc.at[pl.ds(N - ZR_LAST, ZR_LAST)]),
             s)
    plsc.subcore_barrier()

    def body(j, _):
        pltpu.sync_copy(ones_v, acc.at[dst_v.at[j]], add=True)
        return 0

    lax.fori_loop(0, NCHUNK, body, 0)
    plsc.subcore_barrier()
    _rowcopy(lambda b: pltpu.sync_copy(acc.at[pl.ds(b, ZR)],
                                       parts_hbm.at[c, pl.ds(b, ZR)]),
             lambda: pltpu.sync_copy(acc.at[pl.ds(N - ZR_LAST, ZR_LAST)],
                                     parts_hbm.at[c, pl.ds(N - ZR_LAST, ZR_LAST)]),
             s)


HW = H // NC
NCT = 2 * NCHUNK


@functools.partial(
    pl.kernel,
    out_type=jax.ShapeDtypeStruct((NC, N, HW), jnp.float32),
    mesh=_MESH,
    scratch_types=[
        pltpu.VMEM((NCT, K), jnp.int32),
        pltpu.VMEM((NCT, K), jnp.int32),
        pltpu.VMEM((K, HW), jnp.float32),
        pltpu.VMEM((K, HW), jnp.float32),
        pltpu.SemaphoreType.DMA,
        pltpu.SemaphoreType.DMA,
        pltpu.VMEM_SHARED((N, HW), jnp.float32),
    ],
    compiler_params=pltpu.CompilerParams(use_tc_tiling_on_sc=False),
)
def _sc_scatter(g_hbm, src_hbm, dst_hbm, zeros_hbm, parts_hbm,
                src_v, dst_v, rows0, rows1, sem0, sem1, acc):
    c = lax.axis_index("c")
    s = lax.axis_index("s")
    gh = g_hbm.at[c]
    pltpu.sync_copy(src_hbm.at[2 * s], src_v.at[pl.ds(0, NCHUNK)])
    pltpu.sync_copy(src_hbm.at[2 * s + 1], src_v.at[pl.ds(NCHUNK, NCHUNK)])
    pltpu.sync_copy(dst_hbm.at[2 * s], dst_v.at[pl.ds(0, NCHUNK)])
    pltpu.sync_copy(dst_hbm.at[2 * s + 1], dst_v.at[pl.ds(NCHUNK, NCHUNK)])
    _rowcopy(lambda b: pltpu.sync_copy(zeros_hbm.at[pl.ds(b, ZR)],
                                       acc.at[pl.ds(b, ZR)]),
             lambda: pltpu.sync_copy(zeros_hbm.at[pl.ds(N - ZR_LAST, ZR_LAST)],
                                     acc.at[pl.ds(N - ZR_LAST, ZR_LAST)]),
             s)
    plsc.subcore_barrier()

    pltpu.async_copy(gh.at[src_v.at[0]], rows0, sem0)

    def body(j, _):
        @pl.when(j % 2 == 0)
        def _even():
            @pl.when(j + 1 < NCT)
            def _pref():
                pltpu.async_copy(gh.at[src_v.at[j + 1]], rows1, sem1)
            pltpu.make_async_copy(gh.at[src_v.at[j]], rows0, sem0).wait()
            pltpu.sync_copy(rows0, acc.at[dst_v.at[j]], add=True)

        @pl.when(j % 2 == 1)
        def _odd():
            @pl.when(j + 1 < NCT)
            def _pref():
                pltpu.async_copy(gh.at[src_v.at[j + 1]], rows0, sem0)
            pltpu.make_async_copy(gh.at[src_v.at[j]], rows1, sem1).wait()
            pltpu.sync_copy(rows1, acc.at[dst_v.at[j]], add=True)

        return 0

    lax.fori_loop(0, NCT, body, 0)
    plsc.subcore_barrier()
    _rowcopy(lambda b: pltpu.sync_copy(acc.at[pl.ds(b, ZR)],
                                       parts_hbm.at[c, pl.ds(b, ZR)]),
             lambda: pltpu.sync_copy(acc.at[pl.ds(N - ZR_LAST, ZR_LAST)],
                                     parts_hbm.at[c, pl.ds(N - ZR_LAST, ZR_LAST)]),
             s)


@functools.partial(
    pl.kernel,
    out_type=jax.ShapeDtypeStruct((ECP,), jnp.float32),
    mesh=_MESH,
    scratch_types=[
        pltpu.VMEM((N,), jnp.float32),
        pltpu.VMEM((N,), jnp.float32),
        pltpu.VMEM((PPT,), jnp.int32),
        pltpu.VMEM((PPT,), jnp.int32),
        pltpu.VMEM((PPT,), jnp.float32),
    ],
    compiler_params=pltpu.CompilerParams(needs_layout_passes=False),
)
def _sc_pairs(s_hbm, t_hbm, ei0_hbm, ei1_hbm, out_hbm,
              s_v, t_v, i0_v, i1_v, ob_v):
    w = _wid()
    base = pl.multiple_of(w * PPT, 8)
    pltpu.sync_copy(s_hbm, s_v)
    pltpu.sync_copy(t_hbm, t_v)
    pltpu.sync_copy(ei0_hbm.at[pl.ds(base, PPT)], i0_v)
    pltpu.sync_copy(ei1_hbm.at[pl.ds(base, PPT)], i1_v)

    def body(j, _):
        sl = pl.ds(j * 16, 16)
        v0 = plsc.load_gather(s_v, [i0_v[sl]])
        v1 = plsc.load_gather(t_v, [i1_v[sl]])
        z = v0 + v1
        ob_v[sl] = 1.0 / (1.0 + jnp.exp(-z))
        return 0

    lax.fori_loop(0, PCH, body, 0)
    pltpu.sync_copy(ob_v, out_hbm.at[pl.ds(base, PPT)])


BN = 2000


def _dinv_of(deg_ref):
    deg = deg_ref[0, :, 0:1] + deg_ref[1, :, 0:1] + 1.0
    return lax.rsqrt(deg)


_DEG_SPEC = pl.BlockSpec((NC, BN, 16), lambda i: (0, i, 0))


def _split_store(out_ref, g):
    out_ref[0] = g[:, :HW]
    out_ref[1] = g[:, HW:]


def _scale_mm_body(x_ref, w_ref, deg_ref, g_ref):
    xw = jnp.dot(x_ref[...], w_ref[...], preferred_element_type=jnp.float32)
    _split_store(g_ref, _dinv_of(deg_ref) * xw)


def _tc_scale_mm(x, W, deg_parts):
    return pl.pallas_call(
        _scale_mm_body,
        grid=(N // BN,),
        in_specs=[
            pl.BlockSpec((BN, D), lambda i: (i, 0)),
            pl.BlockSpec((D, H), lambda i: (0, 0)),
            _DEG_SPEC,
        ],
        out_specs=pl.BlockSpec((NC, BN, HW), lambda i: (0, i, 0)),
        out_shape=jax.ShapeDtypeStruct((NC, N, HW), jnp.float32),
    )(x, W, deg_parts)


def _layer_body(parts_ref, g_ref, deg_ref, b_ref, w_ref, out_ref):
    dinv = _dinv_of(deg_ref)
    tot = jnp.concatenate([parts_ref[0] + g_ref[0], parts_ref[1] + g_ref[1]],
                          axis=1)
    h = jnp.maximum(dinv * tot + b_ref[...], 0.0)
    hw = jnp.dot(h, w_ref[...], preferred_element_type=jnp.float32)
    _split_store(out_ref, dinv * hw)


def _tc_layer(parts, g, deg_parts, b, W):
    return pl.pallas_call(
        _layer_body,
        grid=(N // BN,),
        in_specs=[
            pl.BlockSpec((NC, BN, HW), lambda i: (0, i, 0)),
            pl.BlockSpec((NC, BN, HW), lambda i: (0, i, 0)),
            _DEG_SPEC,
            pl.BlockSpec((1, H), lambda i: (0, 0)),
            pl.BlockSpec((H, H), lambda i: (0, 0)),
        ],
        out_specs=pl.BlockSpec((NC, BN, HW), lambda i: (0, i, 0)),
        out_shape=jax.ShapeDtypeStruct((NC, N, HW), jnp.float32),
    )(parts, g, deg_parts, b, W)


def _final_body(parts_ref, g_ref, deg_ref, b_ref, wc0_ref, wc1_ref, bc_ref,
                h_ref, s_ref, t_ref):
    dinv = _dinv_of(deg_ref)
    tot = jnp.concatenate([parts_ref[0] + g_ref[0], parts_ref[1] + g_ref[1]],
                          axis=1)
    h = jnp.maximum(dinv * tot + b_ref[...], 0.0)
    h_ref[...] = h
    s_ref[...] = jnp.sum(h * wc0_ref[...], axis=1, keepdims=True) + bc_ref[0]
    t_ref[...] = jnp.sum(h * wc1_ref[...], axis=1, keepdims=True)


def _tc_final(parts, g, deg_parts, b, wc0, wc1, bc):
    return pl.pallas_call(
        _final_body,
        grid=(N // BN,),
        in_specs=[
            pl.BlockSpec((NC, BN, HW), lambda i: (0, i, 0)),
            pl.BlockSpec((NC, BN, HW), lambda i: (0, i, 0)),
            _DEG_SPEC,
            pl.BlockSpec((1, H), lambda i: (0, 0)),
            pl.BlockSpec((1, H), lambda i: (0, 0)),
            pl.BlockSpec((1, H), lambda i: (0, 0)),
            pl.BlockSpec(memory_space=pltpu.SMEM),
        ],
        out_specs=[
            pl.BlockSpec((BN, H), lambda i: (i, 0)),
            pl.BlockSpec((BN, 1), lambda i: (i, 0)),
            pl.BlockSpec((BN, 1), lambda i: (i, 0)),
        ],
        out_shape=[
            jax.ShapeDtypeStruct((N, H), jnp.float32),
            jax.ShapeDtypeStruct((N, 1), jnp.float32),
            jax.ShapeDtypeStruct((N, 1), jnp.float32),
        ],
    )(parts, g, deg_parts, b, wc0, wc1, bc)


def kernel(x, edge_index_ppi, edge_index, W1, b1, W2, b2, Wc, bc):
    src = edge_index_ppi[0].reshape(NW, NCHUNK, K)
    dst = edge_index_ppi[1].reshape(NW, NCHUNK, K)

    ones16 = jnp.ones((K, 16), jnp.float32)
    zeros16 = jnp.zeros((N, 16), jnp.float32)
    zerosHW = jnp.zeros((N, HW), jnp.float32)

    deg_parts = _sc_degree(dst, ones16, zeros16)

    g1 = _tc_scale_mm(x, W1, deg_parts)
    p1 = _sc_scatter(g1, src, dst, zerosHW)
    g2 = _tc_layer(p1, g1, deg_parts, b1.reshape(1, H), W2)
    p2 = _sc_scatter(g2, src, dst, zerosHW)

    wc0 = Wc[:H, 0].reshape(1, H)
    wc1 = Wc[H:, 0].reshape(1, H)
    h2, s_col, t_col = _tc_final(p2, g2, deg_parts, b2.reshape(1, H), wc0,
                                 wc1, bc)

    pad = ECP - EC
    ei0 = jnp.pad(edge_index[0], (0, pad))
    ei1 = jnp.pad(edge_index[1], (0, pad))
    probs = _sc_pairs(s_col.reshape(N), t_col.reshape(N), ei0, ei1)
    return (h2, probs[:EC].reshape(EC, 1))


# speedup vs baseline: 25.9361x; 1.0356x over previous
import functools

import jax
import jax.numpy as jnp
from jax import lax
from jax.experimental import pallas as pl
from jax.experimental.pallas import tpu as pltpu
from jax.experimental.pallas import tpu_sc as plsc

N = 10000
E = 320000
EC = 100000
D = 128
H = 128

NC = 2
NS = 16
NW = NC * NS
EPT = E // NW
K = 125
NCHUNK = EPT // K
RPT = N // NS

PPT = 3136
ECP = PPT * NW
PCH = PPT // 16

ZR = 632
ZR_LAST = N - (NS - 1) * ZR

_MESH = plsc.VectorSubcoreMesh(core_axis_name="c", subcore_axis_name="s",
                               num_cores=NC, num_subcores=NS)


def _wid():
    return lax.axis_index("c") * NS + lax.axis_index("s")


def _rowcopy(fn_main, fn_last, s):
    base = pl.multiple_of(s * ZR, 8)

    @pl.when(s < NS - 1)
    def _():
        fn_main(base)

    @pl.when(s == NS - 1)
    def _():
        fn_last()


@functools.partial(
    pl.kernel,
    out_type=jax.ShapeDtypeStruct((NC, N, 16), jnp.float32),
    mesh=_MESH,
    scratch_types=[
        pltpu.VMEM((NCHUNK, K), jnp.int32),
        pltpu.VMEM((K, 16), jnp.float32),
        pltpu.VMEM_SHARED((N, 16), jnp.float32),
    ],
    compiler_params=pltpu.CompilerParams(use_tc_tiling_on_sc=False),
)
def _sc_degree(dst_hbm, ones_hbm, zeros_hbm, parts_hbm, dst_v, ones_v, acc):
    c = lax.axis_index("c")
    s = lax.axis_index("s")
    w = c * NS + s
    pltpu.sync_copy(dst_hbm.at[w], dst_v)
    pltpu.sync_copy(ones_hbm, ones_v)
    _rowcopy(lambda b: pltpu.sync_copy(zeros_hbm.at[pl.ds(b, ZR)],
                                       acc.at[pl.ds(b, ZR)]),
             lambda: pltpu.sync_copy(zeros_hbm.at[pl.ds(N - ZR_LAST, ZR_LAST)],
                                     acc.at[pl.ds(N - ZR_LAST, ZR_LAST)]),
             s)
    plsc.subcore_barrier()

    def body(j, _):
        pltpu.sync_copy(ones_v, acc.at[dst_v.at[j]], add=True)
        return 0

    lax.fori_loop(0, NCHUNK, body, 0)
    plsc.subcore_barrier()
    _rowcopy(lambda b: pltpu.sync_copy(acc.at[pl.ds(b, ZR)],
                                       parts_hbm.at[c, pl.ds(b, ZR)]),
             lambda: pltpu.sync_copy(acc.at[pl.ds(N - ZR_LAST, ZR_LAST)],
                                     parts_hbm.at[c, pl.ds(N - ZR_LAST, ZR_LAST)]),
             s)


HW = H // NC
NCT = 2 * NCHUNK


NBUF = 5
NGRP = NCT // NBUF
ND = 2
WG = NBUF - ND


@functools.partial(
    pl.kernel,
    out_type=jax.ShapeDtypeStruct((NC, N, HW), jnp.float32),
    mesh=_MESH,
    scratch_types=[
        pltpu.VMEM((NCT, K), jnp.int32),
        pltpu.VMEM((NCT, K), jnp.int32),
        pltpu.VMEM((NBUF, K, HW), jnp.float32),
        pltpu.SemaphoreType.DMA((NBUF,)),
        pltpu.SemaphoreType.DMA((NBUF,)),
        pltpu.VMEM_SHARED((N, HW), jnp.float32),
    ],
    compiler_params=pltpu.CompilerParams(use_tc_tiling_on_sc=False),
)
def _sc_scatter(g_hbm, src_hbm, dst_hbm, zeros_hbm, parts_hbm,
                src_v, dst_v, rows, gsem, ssem, acc):
    c = lax.axis_index("c")
    s = lax.axis_index("s")
    gh = g_hbm.at[c]
    pltpu.sync_copy(src_hbm.at[2 * s], src_v.at[pl.ds(0, NCHUNK)])
    pltpu.sync_copy(src_hbm.at[2 * s + 1], src_v.at[pl.ds(NCHUNK, NCHUNK)])
    pltpu.sync_copy(dst_hbm.at[2 * s], dst_v.at[pl.ds(0, NCHUNK)])
    pltpu.sync_copy(dst_hbm.at[2 * s + 1], dst_v.at[pl.ds(NCHUNK, NCHUNK)])
    _rowcopy(lambda b: pltpu.sync_copy(zeros_hbm.at[pl.ds(b, ZR)],
                                       acc.at[pl.ds(b, ZR)]),
             lambda: pltpu.sync_copy(zeros_hbm.at[pl.ds(N - ZR_LAST, ZR_LAST)],
                                     acc.at[pl.ds(N - ZR_LAST, ZR_LAST)]),
             s)
    plsc.subcore_barrier()

    for b in range(ND):
        pltpu.async_copy(gh.at[src_v.at[b]], rows.at[b], gsem.at[b])

    def body(grp, _):
        for b in range(NBUF):
            i = grp * NBUF + b
            pltpu.make_async_copy(gh.at[src_v.at[i]], rows.at[b],
                                  gsem.at[b]).wait()
            pltpu.async_copy(rows.at[b], acc.at[dst_v.at[i]], ssem.at[b],
                             add=True)
            bg = (b + ND) % NBUF

            def _advance():
                def _drain():
                    pltpu.make_async_copy(rows.at[bg], acc.at[dst_v.at[i]],
                                          ssem.at[bg]).wait()
                if b >= WG:
                    _drain()
                else:
                    pl.when(grp > 0)(_drain)
                pltpu.async_copy(gh.at[src_v.at[i + ND]], rows.at[bg],
                                 gsem.at[bg])

            if b < WG:
                _advance()
            else:
                pl.when(grp < NGRP - 1)(_advance)
        return 0

    lax.fori_loop(0, NGRP, body, 0)
    for b in range(NBUF):
        pltpu.make_async_copy(rows.at[b], acc.at[dst_v.at[0]],
                              ssem.at[b]).wait()
    plsc.subcore_barrier()
    _rowcopy(lambda b: pltpu.sync_copy(acc.at[pl.ds(b, ZR)],
                                       parts_hbm.at[c, pl.ds(b, ZR)]),
             lambda: pltpu.sync_copy(acc.at[pl.ds(N - ZR_LAST, ZR_LAST)],
                                     parts_hbm.at[c, pl.ds(N - ZR_LAST, ZR_LAST)]),
             s)


@functools.partial(
    pl.kernel,
    out_type=jax.ShapeDtypeStruct((ECP,), jnp.float32),
    mesh=_MESH,
    scratch_types=[
        pltpu.VMEM((N,), jnp.float32),
        pltpu.VMEM((N,), jnp.float32),
        pltpu.VMEM((PPT,), jnp.int32),
        pltpu.VMEM((PPT,), jnp.int32),
        pltpu.VMEM((PPT,), jnp.float32),
    ],
    compiler_params=pltpu.CompilerParams(needs_layout_passes=False),
)
def _sc_pairs(s_hbm, t_hbm, ei0_hbm, ei1_hbm, out_hbm,
              s_v, t_v, i0_v, i1_v, ob_v):
    w = _wid()
    base = pl.multiple_of(w * PPT, 8)
    pltpu.sync_copy(s_hbm, s_v)
    pltpu.sync_copy(t_hbm, t_v)
    pltpu.sync_copy(ei0_hbm.at[pl.ds(base, PPT)], i0_v)
    pltpu.sync_copy(ei1_hbm.at[pl.ds(base, PPT)], i1_v)

    def body(j, _):
        sl = pl.ds(j * 16, 16)
        v0 = plsc.load_gather(s_v, [i0_v[sl]])
        v1 = plsc.load_gather(t_v, [i1_v[sl]])
        z = v0 + v1
        ob_v[sl] = 1.0 / (1.0 + jnp.exp(-z))
        return 0

    lax.fori_loop(0, PCH, body, 0)
    pltpu.sync_copy(ob_v, out_hbm.at[pl.ds(base, PPT)])


BN = 2000


def _dinv_of(deg_ref):
    deg = deg_ref[0, :, 0:1] + deg_ref[1, :, 0:1] + 1.0
    return lax.rsqrt(deg)


_DEG_SPEC = pl.BlockSpec((NC, BN, 16), lambda i: (0, i, 0))


def _split_store(out_ref, g):
    out_ref[0] = g[:, :HW]
    out_ref[1] = g[:, HW:]


def _scale_mm_body(x_ref, w_ref, deg_ref, g_ref):
    xw = jnp.dot(x_ref[...], w_ref[...], preferred_element_type=jnp.float32)
    _split_store(g_ref, _dinv_of(deg_ref) * xw)


def _tc_scale_mm(x, W, deg_parts):
    return pl.pallas_call(
        _scale_mm_body,
        grid=(N // BN,),
        in_specs=[
            pl.BlockSpec((BN, D), lambda i: (i, 0)),
            pl.BlockSpec((D, H), lambda i: (0, 0)),
            _DEG_SPEC,
        ],
        out_specs=pl.BlockSpec((NC, BN, HW), lambda i: (0, i, 0)),
        out_shape=jax.ShapeDtypeStruct((NC, N, HW), jnp.float32),
    )(x, W, deg_parts)


def _layer_body(parts_ref, g_ref, deg_ref, b_ref, w_ref, out_ref):
    dinv = _dinv_of(deg_ref)
    tot = jnp.concatenate([parts_ref[0] + g_ref[0], parts_ref[1] + g_ref[1]],
                          axis=1)
    h = jnp.maximum(dinv * tot + b_ref[...], 0.0)
    hw = jnp.dot(h, w_ref[...], preferred_element_type=jnp.float32)
    _split_store(out_ref, dinv * hw)


def _tc_layer(parts, g, deg_parts, b, W):
    return pl.pallas_call(
        _layer_body,
        grid=(N // BN,),
        in_specs=[
            pl.BlockSpec((NC, BN, HW), lambda i: (0, i, 0)),
            pl.BlockSpec((NC, BN, HW), lambda i: (0, i, 0)),
            _DEG_SPEC,
            pl.BlockSpec((1, H), lambda i: (0, 0)),
            pl.BlockSpec((H, H), lambda i: (0, 0)),
        ],
        out_specs=pl.BlockSpec((NC, BN, HW), lambda i: (0, i, 0)),
        out_shape=jax.ShapeDtypeStruct((NC, N, HW), jnp.float32),
    )(parts, g, deg_parts, b, W)


def _final_body(parts_ref, g_ref, deg_ref, b_ref, wc0_ref, wc1_ref, bc_ref,
                h_ref, s_ref, t_ref):
    dinv = _dinv_of(deg_ref)
    tot = jnp.concatenate([parts_ref[0] + g_ref[0], parts_ref[1] + g_ref[1]],
                          axis=1)
    h = jnp.maximum(dinv * tot + b_ref[...], 0.0)
    h_ref[...] = h
    s_ref[...] = jnp.sum(h * wc0_ref[...], axis=1, keepdims=True) + bc_ref[0]
    t_ref[...] = jnp.sum(h * wc1_ref[...], axis=1, keepdims=True)


def _tc_final(parts, g, deg_parts, b, wc0, wc1, bc):
    return pl.pallas_call(
        _final_body,
        grid=(N // BN,),
        in_specs=[
            pl.BlockSpec((NC, BN, HW), lambda i: (0, i, 0)),
            pl.BlockSpec((NC, BN, HW), lambda i: (0, i, 0)),
            _DEG_SPEC,
            pl.BlockSpec((1, H), lambda i: (0, 0)),
            pl.BlockSpec((1, H), lambda i: (0, 0)),
            pl.BlockSpec((1, H), lambda i: (0, 0)),
            pl.BlockSpec(memory_space=pltpu.SMEM),
        ],
        out_specs=[
            pl.BlockSpec((BN, H), lambda i: (i, 0)),
            pl.BlockSpec((BN, 1), lambda i: (i, 0)),
            pl.BlockSpec((BN, 1), lambda i: (i, 0)),
        ],
        out_shape=[
            jax.ShapeDtypeStruct((N, H), jnp.float32),
            jax.ShapeDtypeStruct((N, 1), jnp.float32),
            jax.ShapeDtypeStruct((N, 1), jnp.float32),
        ],
    )(parts, g, deg_parts, b, wc0, wc1, bc)


def kernel(x, edge_index_ppi, edge_index, W1, b1, W2, b2, Wc, bc):
    src = edge_index_ppi[0].reshape(NW, NCHUNK, K)
    dst = edge_index_ppi[1].reshape(NW, NCHUNK, K)

    ones16 = jnp.ones((K, 16), jnp.float32)
    zeros16 = jnp.zeros((N, 16), jnp.float32)
    zerosHW = jnp.zeros((N, HW), jnp.float32)

    deg_parts = _sc_degree(dst, ones16, zeros16)

    g1 = _tc_scale_mm(x, W1, deg_parts)
    p1 = _sc_scatter(g1, src, dst, zerosHW)
    g2 = _tc_layer(p1, g1, deg_parts, b1.reshape(1, H), W2)
    p2 = _sc_scatter(g2, src, dst, zerosHW)

    wc0 = Wc[:H, 0].reshape(1, H)
    wc1 = Wc[H:, 0].reshape(1, H)
    h2, s_col, t_col = _tc_final(p2, g2, deg_parts, b2.reshape(1, H), wc0,
                                 wc1, bc)

    pad = ECP - EC
    ei0 = jnp.pad(edge_index[0], (0, pad))
    ei1 = jnp.pad(edge_index[1], (0, pad))
    probs = _sc_pairs(s_col.reshape(N), t_col.reshape(N), ei0, ei1)
    return (h2, probs[:EC].reshape(EC, 1))


# speedup vs baseline: 28.7636x; 1.1090x over previous
import functools

import jax
import jax.numpy as jnp
from jax import lax
from jax.experimental import pallas as pl
from jax.experimental.pallas import tpu as pltpu
from jax.experimental.pallas import tpu_sc as plsc

N = 10000
E = 320000
EC = 100000
D = 128
H = 128

NC = 2
NS = 16
NW = NC * NS
EPT = E // NW
K = 125
NCHUNK = EPT // K
RPT = N // NS

PPT = 3136
ECP = PPT * NW
PCH = PPT // 16

ZR = 632
ZR_LAST = N - (NS - 1) * ZR

_MESH = plsc.VectorSubcoreMesh(core_axis_name="c", subcore_axis_name="s",
                               num_cores=NC, num_subcores=NS)


def _wid():
    return lax.axis_index("c") * NS + lax.axis_index("s")


def _rowcopy(fn_main, fn_last, s):
    base = pl.multiple_of(s * ZR, 8)

    @pl.when(s < NS - 1)
    def _():
        fn_main(base)

    @pl.when(s == NS - 1)
    def _():
        fn_last()


@functools.partial(
    pl.kernel,
    out_type=jax.ShapeDtypeStruct((NC, N, 16), jnp.float32),
    mesh=_MESH,
    scratch_types=[
        pltpu.VMEM((NCHUNK, K), jnp.int32),
        pltpu.VMEM((K, 16), jnp.float32),
        pltpu.VMEM_SHARED((N, 16), jnp.float32),
    ],
    compiler_params=pltpu.CompilerParams(use_tc_tiling_on_sc=False),
)
def _sc_degree(dst_hbm, ones_hbm, zeros_hbm, parts_hbm, dst_v, ones_v, acc):
    c = lax.axis_index("c")
    s = lax.axis_index("s")
    w = c * NS + s
    pltpu.sync_copy(dst_hbm.at[w], dst_v)
    pltpu.sync_copy(ones_hbm, ones_v)
    _rowcopy(lambda b: pltpu.sync_copy(zeros_hbm.at[pl.ds(b, ZR)],
                                       acc.at[pl.ds(b, ZR)]),
             lambda: pltpu.sync_copy(zeros_hbm.at[pl.ds(N - ZR_LAST, ZR_LAST)],
                                     acc.at[pl.ds(N - ZR_LAST, ZR_LAST)]),
             s)
    plsc.subcore_barrier()

    def body(j, _):
        pltpu.sync_copy(ones_v, acc.at[dst_v.at[j]], add=True)
        return 0

    lax.fori_loop(0, NCHUNK, body, 0)
    plsc.subcore_barrier()
    _rowcopy(lambda b: pltpu.sync_copy(acc.at[pl.ds(b, ZR)],
                                       parts_hbm.at[c, pl.ds(b, ZR)]),
             lambda: pltpu.sync_copy(acc.at[pl.ds(N - ZR_LAST, ZR_LAST)],
                                     parts_hbm.at[c, pl.ds(N - ZR_LAST, ZR_LAST)]),
             s)


HW = H // NC
NCT = 2 * NCHUNK


NBUF = 5
NGRP = NCT // NBUF
ND = 3
WG = NBUF - ND


@functools.partial(
    pl.kernel,
    out_type=jax.ShapeDtypeStruct((NC, N, HW), jnp.float32),
    mesh=_MESH,
    scratch_types=[
        pltpu.VMEM((NCT, K), jnp.int32),
        pltpu.VMEM((NCT, K), jnp.int32),
        pltpu.VMEM((NBUF, K, HW), jnp.float32),
        pltpu.SemaphoreType.DMA((NBUF,)),
        pltpu.SemaphoreType.DMA((NBUF,)),
        pltpu.VMEM_SHARED((N, HW), jnp.float32),
    ],
    compiler_params=pltpu.CompilerParams(use_tc_tiling_on_sc=False),
)
def _sc_scatter(g_hbm, src_hbm, dst_hbm, zeros_hbm, parts_hbm,
                src_v, dst_v, rows, gsem, ssem, acc):
    c = lax.axis_index("c")
    s = lax.axis_index("s")
    gh = g_hbm.at[c]
    pltpu.sync_copy(src_hbm.at[2 * s], src_v.at[pl.ds(0, NCHUNK)])
    pltpu.sync_copy(src_hbm.at[2 * s + 1], src_v.at[pl.ds(NCHUNK, NCHUNK)])
    pltpu.sync_copy(dst_hbm.at[2 * s], dst_v.at[pl.ds(0, NCHUNK)])
    pltpu.sync_copy(dst_hbm.at[2 * s + 1], dst_v.at[pl.ds(NCHUNK, NCHUNK)])
    _rowcopy(lambda b: pltpu.sync_copy(zeros_hbm.at[pl.ds(b, ZR)],
                                       acc.at[pl.ds(b, ZR)]),
             lambda: pltpu.sync_copy(zeros_hbm.at[pl.ds(N - ZR_LAST, ZR_LAST)],
                                     acc.at[pl.ds(N - ZR_LAST, ZR_LAST)]),
             s)
    plsc.subcore_barrier()

    for b in range(ND):
        pltpu.async_copy(gh.at[src_v.at[b]], rows.at[b], gsem.at[b])

    def body(grp, _):
        for b in range(NBUF):
            i = grp * NBUF + b
            pltpu.make_async_copy(gh.at[src_v.at[i]], rows.at[b],
                                  gsem.at[b]).wait()
            pltpu.async_copy(rows.at[b], acc.at[dst_v.at[i]], ssem.at[b],
                             add=True)
            bg = (b + ND) % NBUF

            def _advance():
                def _drain():
                    pltpu.make_async_copy(rows.at[bg], acc.at[dst_v.at[i]],
                                          ssem.at[bg]).wait()
                if b >= WG:
                    _drain()
                else:
                    pl.when(grp > 0)(_drain)
                pltpu.async_copy(gh.at[src_v.at[i + ND]], rows.at[bg],
                                 gsem.at[bg])

            if b < WG:
                _advance()
            else:
                pl.when(grp < NGRP - 1)(_advance)
        return 0

    lax.fori_loop(0, NGRP, body, 0)
    for b in range(NBUF):
        pltpu.make_async_copy(rows.at[b], acc.at[dst_v.at[0]],
                              ssem.at[b]).wait()
    plsc.subcore_barrier()
    _rowcopy(lambda b: pltpu.sync_copy(acc.at[pl.ds(b, ZR)],
                                       parts_hbm.at[c, pl.ds(b, ZR)]),
             lambda: pltpu.sync_copy(acc.at[pl.ds(N - ZR_LAST, ZR_LAST)],
                                     parts_hbm.at[c, pl.ds(N - ZR_LAST, ZR_LAST)]),
             s)


@functools.partial(
    pl.kernel,
    out_type=jax.ShapeDtypeStruct((ECP,), jnp.float32),
    mesh=_MESH,
    scratch_types=[
        pltpu.VMEM((N,), jnp.float32),
        pltpu.VMEM((N,), jnp.float32),
        pltpu.VMEM((PPT,), jnp.int32),
        pltpu.VMEM((PPT,), jnp.int32),
        pltpu.VMEM((PPT,), jnp.float32),
    ],
    compiler_params=pltpu.CompilerParams(needs_layout_passes=False),
)
def _sc_pairs(s_hbm, t_hbm, ei0_hbm, ei1_hbm, out_hbm,
              s_v, t_v, i0_v, i1_v, ob_v):
    w = _wid()
    base = pl.multiple_of(w * PPT, 8)
    pltpu.sync_copy(s_hbm, s_v)
    pltpu.sync_copy(t_hbm, t_v)
    pltpu.sync_copy(ei0_hbm.at[pl.ds(base, PPT)], i0_v)
    pltpu.sync_copy(ei1_hbm.at[pl.ds(base, PPT)], i1_v)

    def body(j, _):
        sl = pl.ds(j * 16, 16)
        v0 = plsc.load_gather(s_v, [i0_v[sl]])
        v1 = plsc.load_gather(t_v, [i1_v[sl]])
        z = v0 + v1
        ob_v[sl] = 1.0 / (1.0 + jnp.exp(-z))
        return 0

    lax.fori_loop(0, PCH, body, 0)
    pltpu.sync_copy(ob_v, out_hbm.at[pl.ds(base, PPT)])


BN = 2000


def _dinv_of(deg_ref):
    deg = deg_ref[0, :, 0:1] + deg_ref[1, :, 0:1] + 1.0
    return lax.rsqrt(deg)


_DEG_SPEC = pl.BlockSpec((NC, BN, 16), lambda i: (0, i, 0))


def _split_store(out_ref, g):
    out_ref[0] = g[:, :HW]
    out_ref[1] = g[:, HW:]


def _scale_mm_body(x_ref, w_ref, deg_ref, g_ref):
    xw = jnp.dot(x_ref[...], w_ref[...], preferred_element_type=jnp.float32)
    _split_store(g_ref, _dinv_of(deg_ref) * xw)


def _tc_scale_mm(x, W, deg_parts):
    return pl.pallas_call(
        _scale_mm_body,
        grid=(N // BN,),
        in_specs=[
            pl.BlockSpec((BN, D), lambda i: (i, 0)),
            pl.BlockSpec((D, H), lambda i: (0, 0)),
            _DEG_SPEC,
        ],
        out_specs=pl.BlockSpec((NC, BN, HW), lambda i: (0, i, 0)),
        out_shape=jax.ShapeDtypeStruct((NC, N, HW), jnp.float32),
    )(x, W, deg_parts)


def _layer_body(parts_ref, g_ref, deg_ref, b_ref, w_ref, out_ref):
    dinv = _dinv_of(deg_ref)
    tot = jnp.concatenate([parts_ref[0] + g_ref[0], parts_ref[1] + g_ref[1]],
                          axis=1)
    h = jnp.maximum(dinv * tot + b_ref[...], 0.0)
    hw = jnp.dot(h, w_ref[...], preferred_element_type=jnp.float32)
    _split_store(out_ref, dinv * hw)


def _tc_layer(parts, g, deg_parts, b, W):
    return pl.pallas_call(
        _layer_body,
        grid=(N // BN,),
        in_specs=[
            pl.BlockSpec((NC, BN, HW), lambda i: (0, i, 0)),
            pl.BlockSpec((NC, BN, HW), lambda i: (0, i, 0)),
            _DEG_SPEC,
            pl.BlockSpec((1, H), lambda i: (0, 0)),
            pl.BlockSpec((H, H), lambda i: (0, 0)),
        ],
        out_specs=pl.BlockSpec((NC, BN, HW), lambda i: (0, i, 0)),
        out_shape=jax.ShapeDtypeStruct((NC, N, HW), jnp.float32),
    )(parts, g, deg_parts, b, W)


def _final_body(parts_ref, g_ref, deg_ref, b_ref, wc0_ref, wc1_ref, bc_ref,
                h_ref, s_ref, t_ref):
    dinv = _dinv_of(deg_ref)
    tot = jnp.concatenate([parts_ref[0] + g_ref[0], parts_ref[1] + g_ref[1]],
                          axis=1)
    h = jnp.maximum(dinv * tot + b_ref[...], 0.0)
    h_ref[...] = h
    s_ref[...] = jnp.sum(h * wc0_ref[...], axis=1, keepdims=True) + bc_ref[0]
    t_ref[...] = jnp.sum(h * wc1_ref[...], axis=1, keepdims=True)


def _tc_final(parts, g, deg_parts, b, wc0, wc1, bc):
    return pl.pallas_call(
        _final_body,
        grid=(N // BN,),
        in_specs=[
            pl.BlockSpec((NC, BN, HW), lambda i: (0, i, 0)),
            pl.BlockSpec((NC, BN, HW), lambda i: (0, i, 0)),
            _DEG_SPEC,
            pl.BlockSpec((1, H), lambda i: (0, 0)),
            pl.BlockSpec((1, H), lambda i: (0, 0)),
            pl.BlockSpec((1, H), lambda i: (0, 0)),
            pl.BlockSpec(memory_space=pltpu.SMEM),
        ],
        out_specs=[
            pl.BlockSpec((BN, H), lambda i: (i, 0)),
            pl.BlockSpec((BN, 1), lambda i: (i, 0)),
            pl.BlockSpec((BN, 1), lambda i: (i, 0)),
        ],
        out_shape=[
            jax.ShapeDtypeStruct((N, H), jnp.float32),
            jax.ShapeDtypeStruct((N, 1), jnp.float32),
            jax.ShapeDtypeStruct((N, 1), jnp.float32),
        ],
    )(parts, g, deg_parts, b, wc0, wc1, bc)


def kernel(x, edge_index_ppi, edge_index, W1, b1, W2, b2, Wc, bc):
    src = edge_index_ppi[0].reshape(NW, NCHUNK, K)
    dst = edge_index_ppi[1].reshape(NW, NCHUNK, K)

    ones16 = jnp.ones((K, 16), jnp.float32)
    zeros16 = jnp.zeros((N, 16), jnp.float32)
    zerosHW = jnp.zeros((N, HW), jnp.float32)

    deg_parts = _sc_degree(dst, ones16, zeros16)

    g1 = _tc_scale_mm(x, W1, deg_parts)
    p1 = _sc_scatter(g1, src, dst, zerosHW)
    g2 = _tc_layer(p1, g1, deg_parts, b1.reshape(1, H), W2)
    p2 = _sc_scatter(g2, src, dst, zerosHW)

    wc0 = Wc[:H, 0].reshape(1, H)
    wc1 = Wc[H:, 0].reshape(1, H)
    h2, s_col, t_col = _tc_final(p2, g2, deg_parts, b2.reshape(1, H), wc0,
                                 wc1, bc)

    pad = ECP - EC
    ei0 = jnp.pad(edge_index[0], (0, pad))
    ei1 = jnp.pad(edge_index[1], (0, pad))
    probs = _sc_pairs(s_col.reshape(N), t_col.reshape(N), ei0, ei1)
    return (h2, probs[:EC].reshape(EC, 1))


# speedup vs baseline: 29.3366x; 1.0199x over previous
import functools

import jax
import jax.numpy as jnp
from jax import lax
from jax.experimental import pallas as pl
from jax.experimental.pallas import tpu as pltpu
from jax.experimental.pallas import tpu_sc as plsc

N = 10000
E = 320000
EC = 100000
D = 128
H = 128

NC = 2
NS = 16
NW = NC * NS
EPT = E // NW
K = 125
NCHUNK = EPT // K
RPT = N // NS

PPT = 3136
ECP = PPT * NW
PCH = PPT // 16

ZR = 632
ZR_LAST = N - (NS - 1) * ZR

_MESH = plsc.VectorSubcoreMesh(core_axis_name="c", subcore_axis_name="s",
                               num_cores=NC, num_subcores=NS)


def _wid():
    return lax.axis_index("c") * NS + lax.axis_index("s")


def _rowcopy(fn_main, fn_last, s):
    base = pl.multiple_of(s * ZR, 8)

    @pl.when(s < NS - 1)
    def _():
        fn_main(base)

    @pl.when(s == NS - 1)
    def _():
        fn_last()


@functools.partial(
    pl.kernel,
    out_type=jax.ShapeDtypeStruct((NC, N, 16), jnp.float32),
    mesh=_MESH,
    scratch_types=[
        pltpu.VMEM((NCHUNK, K), jnp.int32),
        pltpu.VMEM((K, 16), jnp.float32),
        pltpu.VMEM_SHARED((N, 16), jnp.float32),
    ],
    compiler_params=pltpu.CompilerParams(use_tc_tiling_on_sc=False),
)
def _sc_degree(dst_hbm, ones_hbm, zeros_hbm, parts_hbm, dst_v, ones_v, acc):
    c = lax.axis_index("c")
    s = lax.axis_index("s")
    w = c * NS + s
    pltpu.sync_copy(dst_hbm.at[w], dst_v)
    pltpu.sync_copy(ones_hbm, ones_v)
    _rowcopy(lambda b: pltpu.sync_copy(zeros_hbm.at[pl.ds(b, ZR)],
                                       acc.at[pl.ds(b, ZR)]),
             lambda: pltpu.sync_copy(zeros_hbm.at[pl.ds(N - ZR_LAST, ZR_LAST)],
                                     acc.at[pl.ds(N - ZR_LAST, ZR_LAST)]),
             s)
    plsc.subcore_barrier()

    def body(j, _):
        pltpu.sync_copy(ones_v, acc.at[dst_v.at[j]], add=True)
        return 0

    lax.fori_loop(0, NCHUNK, body, 0)
    plsc.subcore_barrier()
    _rowcopy(lambda b: pltpu.sync_copy(acc.at[pl.ds(b, ZR)],
                                       parts_hbm.at[c, pl.ds(b, ZR)]),
             lambda: pltpu.sync_copy(acc.at[pl.ds(N - ZR_LAST, ZR_LAST)],
                                     parts_hbm.at[c, pl.ds(N - ZR_LAST, ZR_LAST)]),
             s)


HW = H // NC
NCT = 2 * NCHUNK


NBUF = 5
NGRP = NCT // NBUF
ND = 4
WG = NBUF - ND


@functools.partial(
    pl.kernel,
    out_type=jax.ShapeDtypeStruct((NC, N, HW), jnp.float32),
    mesh=_MESH,
    scratch_types=[
        pltpu.VMEM((NCT, K), jnp.int32),
        pltpu.VMEM((NCT, K), jnp.int32),
        pltpu.VMEM((NBUF, K, HW), jnp.float32),
        pltpu.SemaphoreType.DMA((NBUF,)),
        pltpu.SemaphoreType.DMA((NBUF,)),
        pltpu.VMEM_SHARED((N, HW), jnp.float32),
    ],
    compiler_params=pltpu.CompilerParams(use_tc_tiling_on_sc=False),
)
def _sc_scatter(g_hbm, src_hbm, dst_hbm, zeros_hbm, parts_hbm,
                src_v, dst_v, rows, gsem, ssem, acc):
    c = lax.axis_index("c")
    s = lax.axis_index("s")
    gh = g_hbm.at[c]
    pltpu.sync_copy(src_hbm.at[2 * s], src_v.at[pl.ds(0, NCHUNK)])
    pltpu.sync_copy(src_hbm.at[2 * s + 1], src_v.at[pl.ds(NCHUNK, NCHUNK)])
    pltpu.sync_copy(dst_hbm.at[2 * s], dst_v.at[pl.ds(0, NCHUNK)])
    pltpu.sync_copy(dst_hbm.at[2 * s + 1], dst_v.at[pl.ds(NCHUNK, NCHUNK)])
    _rowcopy(lambda b: pltpu.sync_copy(zeros_hbm.at[pl.ds(b, ZR)],
                                       acc.at[pl.ds(b, ZR)]),
             lambda: pltpu.sync_copy(zeros_hbm.at[pl.ds(N - ZR_LAST, ZR_LAST)],
                                     acc.at[pl.ds(N - ZR_LAST, ZR_LAST)]),
             s)
    plsc.subcore_barrier()

    for b in range(ND):
        pltpu.async_copy(gh.at[src_v.at[b]], rows.at[b], gsem.at[b])

    def body(grp, _):
        for b in range(NBUF):
            i = grp * NBUF + b
            pltpu.make_async_copy(gh.at[src_v.at[i]], rows.at[b],
                                  gsem.at[b]).wait()
            pltpu.async_copy(rows.at[b], acc.at[dst_v.at[i]], ssem.at[b],
                             add=True)
            bg = (b + ND) % NBUF

            def _advance():
                def _drain():
                    pltpu.make_async_copy(rows.at[bg], acc.at[dst_v.at[i]],
                                          ssem.at[bg]).wait()
                if b >= WG:
                    _drain()
                else:
                    pl.when(grp > 0)(_drain)
                pltpu.async_copy(gh.at[src_v.at[i + ND]], rows.at[bg],
                                 gsem.at[bg])

            if b < WG:
                _advance()
            else:
                pl.when(grp < NGRP - 1)(_advance)
        return 0

    lax.fori_loop(0, NGRP, body, 0)
    for b in range(NBUF):
        pltpu.make_async_copy(rows.at[b], acc.at[dst_v.at[0]],
                              ssem.at[b]).wait()
    plsc.subcore_barrier()
    _rowcopy(lambda b: pltpu.sync_copy(acc.at[pl.ds(b, ZR)],
                                       parts_hbm.at[c, pl.ds(b, ZR)]),
             lambda: pltpu.sync_copy(acc.at[pl.ds(N - ZR_LAST, ZR_LAST)],
                                     parts_hbm.at[c, pl.ds(N - ZR_LAST, ZR_LAST)]),
             s)


@functools.partial(
    pl.kernel,
    out_type=jax.ShapeDtypeStruct((ECP,), jnp.float32),
    mesh=_MESH,
    scratch_types=[
        pltpu.VMEM((N,), jnp.float32),
        pltpu.VMEM((N,), jnp.float32),
        pltpu.VMEM((PPT,), jnp.int32),
        pltpu.VMEM((PPT,), jnp.int32),
        pltpu.VMEM((PPT,), jnp.float32),
    ],
    compiler_params=pltpu.CompilerParams(needs_layout_passes=False),
)
def _sc_pairs(s_hbm, t_hbm, ei0_hbm, ei1_hbm, out_hbm,
              s_v, t_v, i0_v, i1_v, ob_v):
    w = _wid()
    base = pl.multiple_of(w * PPT, 8)
    pltpu.sync_copy(s_hbm, s_v)
    pltpu.sync_copy(t_hbm, t_v)
    pltpu.sync_copy(ei0_hbm.at[pl.ds(base, PPT)], i0_v)
    pltpu.sync_copy(ei1_hbm.at[pl.ds(base, PPT)], i1_v)

    def body(j, _):
        sl = pl.ds(j * 16, 16)
        v0 = plsc.load_gather(s_v, [i0_v[sl]])
        v1 = plsc.load_gather(t_v, [i1_v[sl]])
        z = v0 + v1
        ob_v[sl] = 1.0 / (1.0 + jnp.exp(-z))
        return 0

    lax.fori_loop(0, PCH, body, 0)
    pltpu.sync_copy(ob_v, out_hbm.at[pl.ds(base, PPT)])


BN = 2000


def _dinv_of(deg_ref):
    deg = deg_ref[0, :, 0:1] + deg_ref[1, :, 0:1] + 1.0
    return lax.rsqrt(deg)


_DEG_SPEC = pl.BlockSpec((NC, BN, 16), lambda i: (0, i, 0))


def _split_store(out_ref, g):
    out_ref[0] = g[:, :HW]
    out_ref[1] = g[:, HW:]


def _scale_mm_body(x_ref, w_ref, deg_ref, g_ref):
    xw = jnp.dot(x_ref[...], w_ref[...], preferred_element_type=jnp.float32)
    _split_store(g_ref, _dinv_of(deg_ref) * xw)


def _tc_scale_mm(x, W, deg_parts):
    return pl.pallas_call(
        _scale_mm_body,
        grid=(N // BN,),
        in_specs=[
            pl.BlockSpec((BN, D), lambda i: (i, 0)),
            pl.BlockSpec((D, H), lambda i: (0, 0)),
            _DEG_SPEC,
        ],
        out_specs=pl.BlockSpec((NC, BN, HW), lambda i: (0, i, 0)),
        out_shape=jax.ShapeDtypeStruct((NC, N, HW), jnp.float32),
    )(x, W, deg_parts)


def _layer_body(parts_ref, g_ref, deg_ref, b_ref, w_ref, out_ref):
    dinv = _dinv_of(deg_ref)
    tot = jnp.concatenate([parts_ref[0] + g_ref[0], parts_ref[1] + g_ref[1]],
                          axis=1)
    h = jnp.maximum(dinv * tot + b_ref[...], 0.0)
    hw = jnp.dot(h, w_ref[...], preferred_element_type=jnp.float32)
    _split_store(out_ref, dinv * hw)


def _tc_layer(parts, g, deg_parts, b, W):
    return pl.pallas_call(
        _layer_body,
        grid=(N // BN,),
        in_specs=[
            pl.BlockSpec((NC, BN, HW), lambda i: (0, i, 0)),
            pl.BlockSpec((NC, BN, HW), lambda i: (0, i, 0)),
            _DEG_SPEC,
            pl.BlockSpec((1, H), lambda i: (0, 0)),
            pl.BlockSpec((H, H), lambda i: (0, 0)),
        ],
        out_specs=pl.BlockSpec((NC, BN, HW), lambda i: (0, i, 0)),
        out_shape=jax.ShapeDtypeStruct((NC, N, HW), jnp.float32),
    )(parts, g, deg_parts, b, W)


def _final_body(parts_ref, g_ref, deg_ref, b_ref, wc0_ref, wc1_ref, bc_ref,
                h_ref, s_ref, t_ref):
    dinv = _dinv_of(deg_ref)
    tot = jnp.concatenate([parts_ref[0] + g_ref[0], parts_ref[1] + g_ref[1]],
                          axis=1)
    h = jnp.maximum(dinv * tot + b_ref[...], 0.0)
    h_ref[...] = h
    s_ref[...] = jnp.sum(h * wc0_ref[...], axis=1, keepdims=True) + bc_ref[0]
    t_ref[...] = jnp.sum(h * wc1_ref[...], axis=1, keepdims=True)


def _tc_final(parts, g, deg_parts, b, wc0, wc1, bc):
    return pl.pallas_call(
        _final_body,
        grid=(N // BN,),
        in_specs=[
            pl.BlockSpec((NC, BN, HW), lambda i: (0, i, 0)),
            pl.BlockSpec((NC, BN, HW), lambda i: (0, i, 0)),
            _DEG_SPEC,
            pl.BlockSpec((1, H), lambda i: (0, 0)),
            pl.BlockSpec((1, H), lambda i: (0, 0)),
            pl.BlockSpec((1, H), lambda i: (0, 0)),
            pl.BlockSpec(memory_space=pltpu.SMEM),
        ],
        out_specs=[
            pl.BlockSpec((BN, H), lambda i: (i, 0)),
            pl.BlockSpec((BN, 1), lambda i: (i, 0)),
            pl.BlockSpec((BN, 1), lambda i: (i, 0)),
        ],
        out_shape=[
            jax.ShapeDtypeStruct((N, H), jnp.float32),
            jax.ShapeDtypeStruct((N, 1), jnp.float32),
            jax.ShapeDtypeStruct((N, 1), jnp.float32),
        ],
    )(parts, g, deg_parts, b, wc0, wc1, bc)


def kernel(x, edge_index_ppi, edge_index, W1, b1, W2, b2, Wc, bc):
    src = edge_index_ppi[0].reshape(NW, NCHUNK, K)
    dst = edge_index_ppi[1].reshape(NW, NCHUNK, K)

    ones16 = jnp.ones((K, 16), jnp.float32)
    zeros16 = jnp.zeros((N, 16), jnp.float32)
    zerosHW = jnp.zeros((N, HW), jnp.float32)

    deg_parts = _sc_degree(dst, ones16, zeros16)

    g1 = _tc_scale_mm(x, W1, deg_parts)
    p1 = _sc_scatter(g1, src, dst, zerosHW)
    g2 = _tc_layer(p1, g1, deg_parts, b1.reshape(1, H), W2)
    p2 = _sc_scatter(g2, src, dst, zerosHW)

    wc0 = Wc[:H, 0].reshape(1, H)
    wc1 = Wc[H:, 0].reshape(1, H)
    h2, s_col, t_col = _tc_final(p2, g2, deg_parts, b2.reshape(1, H), wc0,
                                 wc1, bc)

    pad = ECP - EC
    ei0 = jnp.pad(edge_index[0], (0, pad))
    ei1 = jnp.pad(edge_index[1], (0, pad))
    probs = _sc_pairs(s_col.reshape(N), t_col.reshape(N), ei0, ei1)
    return (h2, probs[:EC].reshape(EC, 1))


# speedup vs baseline: 29.6506x; 1.0107x over previous
import functools

import jax
import jax.numpy as jnp
from jax import lax
from jax.experimental import pallas as pl
from jax.experimental.pallas import tpu as pltpu
from jax.experimental.pallas import tpu_sc as plsc

N = 10000
E = 320000
EC = 100000
D = 128
H = 128

NC = 2
NS = 16
NW = NC * NS
EPT = E // NW
K = 125
NCHUNK = EPT // K
RPT = N // NS

PPT = 3136
ECP = PPT * NW
PCH = PPT // 16

ZR = 632
ZR_LAST = N - (NS - 1) * ZR

_MESH = plsc.VectorSubcoreMesh(core_axis_name="c", subcore_axis_name="s",
                               num_cores=NC, num_subcores=NS)


def _wid():
    return lax.axis_index("c") * NS + lax.axis_index("s")


def _rowcopy(fn_main, fn_last, s):
    base = pl.multiple_of(s * ZR, 8)

    @pl.when(s < NS - 1)
    def _():
        fn_main(base)

    @pl.when(s == NS - 1)
    def _():
        fn_last()


@functools.partial(
    pl.kernel,
    out_type=jax.ShapeDtypeStruct((NC, N, 16), jnp.float32),
    mesh=_MESH,
    scratch_types=[
        pltpu.VMEM((NCHUNK, K), jnp.int32),
        pltpu.VMEM((K, 16), jnp.float32),
        pltpu.SemaphoreType.DMA,
        pltpu.VMEM_SHARED((N, 16), jnp.float32),
    ],
    compiler_params=pltpu.CompilerParams(use_tc_tiling_on_sc=False),
)
def _sc_degree(dst_hbm, ones_hbm, zeros_hbm, parts_hbm, dst_v, ones_v, sem,
               acc):
    c = lax.axis_index("c")
    s = lax.axis_index("s")
    w = c * NS + s
    pltpu.sync_copy(dst_hbm.at[w], dst_v)
    pltpu.sync_copy(ones_hbm, ones_v)
    _rowcopy(lambda b: pltpu.sync_copy(zeros_hbm.at[pl.ds(b, ZR)],
                                       acc.at[pl.ds(b, ZR)]),
             lambda: pltpu.sync_copy(zeros_hbm.at[pl.ds(N - ZR_LAST, ZR_LAST)],
                                     acc.at[pl.ds(N - ZR_LAST, ZR_LAST)]),
             s)
    plsc.subcore_barrier()

    def body(j, _):
        pltpu.async_copy(ones_v, acc.at[dst_v.at[j]], sem, add=True)
        return 0

    lax.fori_loop(0, NCHUNK, body, 0)

    def drain(j, _):
        pltpu.make_async_copy(ones_v, acc.at[dst_v.at[0]], sem).wait()
        return 0

    lax.fori_loop(0, NCHUNK, drain, 0)
    plsc.subcore_barrier()
    _rowcopy(lambda b: pltpu.sync_copy(acc.at[pl.ds(b, ZR)],
                                       parts_hbm.at[c, pl.ds(b, ZR)]),
             lambda: pltpu.sync_copy(acc.at[pl.ds(N - ZR_LAST, ZR_LAST)],
                                     parts_hbm.at[c, pl.ds(N - ZR_LAST, ZR_LAST)]),
             s)


HW = H // NC
NCT = 2 * NCHUNK


NBUF = 5
NGRP = NCT // NBUF
ND = 4
WG = NBUF - ND


@functools.partial(
    pl.kernel,
    out_type=jax.ShapeDtypeStruct((NC, N, HW), jnp.float32),
    mesh=_MESH,
    scratch_types=[
        pltpu.VMEM((NCT, K), jnp.int32),
        pltpu.VMEM((NCT, K), jnp.int32),
        pltpu.VMEM((NBUF, K, HW), jnp.float32),
        pltpu.SemaphoreType.DMA((NBUF,)),
        pltpu.SemaphoreType.DMA((NBUF,)),
        pltpu.VMEM_SHARED((N, HW), jnp.float32),
    ],
    compiler_params=pltpu.CompilerParams(use_tc_tiling_on_sc=False),
)
def _sc_scatter(g_hbm, src_hbm, dst_hbm, zeros_hbm, parts_hbm,
                src_v, dst_v, rows, gsem, ssem, acc):
    c = lax.axis_index("c")
    s = lax.axis_index("s")
    gh = g_hbm.at[c]
    pltpu.sync_copy(src_hbm.at[2 * s], src_v.at[pl.ds(0, NCHUNK)])
    pltpu.sync_copy(src_hbm.at[2 * s + 1], src_v.at[pl.ds(NCHUNK, NCHUNK)])
    pltpu.sync_copy(dst_hbm.at[2 * s], dst_v.at[pl.ds(0, NCHUNK)])
    pltpu.sync_copy(dst_hbm.at[2 * s + 1], dst_v.at[pl.ds(NCHUNK, NCHUNK)])
    _rowcopy(lambda b: pltpu.sync_copy(zeros_hbm.at[pl.ds(b, ZR)],
                                       acc.at[pl.ds(b, ZR)]),
             lambda: pltpu.sync_copy(zeros_hbm.at[pl.ds(N - ZR_LAST, ZR_LAST)],
                                     acc.at[pl.ds(N - ZR_LAST, ZR_LAST)]),
             s)
    plsc.subcore_barrier()

    for b in range(ND):
        pltpu.async_copy(gh.at[src_v.at[b]], rows.at[b], gsem.at[b])

    def body(grp, _):
        for b in range(NBUF):
            i = grp * NBUF + b
            pltpu.make_async_copy(gh.at[src_v.at[i]], rows.at[b],
                                  gsem.at[b]).wait()
            pltpu.async_copy(rows.at[b], acc.at[dst_v.at[i]], ssem.at[b],
                             add=True)
            bg = (b + ND) % NBUF

            def _advance():
                def _drain():
                    pltpu.make_async_copy(rows.at[bg], acc.at[dst_v.at[i]],
                                          ssem.at[bg]).wait()
                if b >= WG:
                    _drain()
                else:
                    pl.when(grp > 0)(_drain)
                pltpu.async_copy(gh.at[src_v.at[i + ND]], rows.at[bg],
                                 gsem.at[bg])

            if b < WG:
                _advance()
            else:
                pl.when(grp < NGRP - 1)(_advance)
        return 0

    lax.fori_loop(0, NGRP, body, 0)
    for b in range(NBUF):
        pltpu.make_async_copy(rows.at[b], acc.at[dst_v.at[0]],
                              ssem.at[b]).wait()
    plsc.subcore_barrier()
    _rowcopy(lambda b: pltpu.sync_copy(acc.at[pl.ds(b, ZR)],
                                       parts_hbm.at[c, pl.ds(b, ZR)]),
             lambda: pltpu.sync_copy(acc.at[pl.ds(N - ZR_LAST, ZR_LAST)],
                                     parts_hbm.at[c, pl.ds(N - ZR_LAST, ZR_LAST)]),
             s)


@functools.partial(
    pl.kernel,
    out_type=jax.ShapeDtypeStruct((ECP,), jnp.float32),
    mesh=_MESH,
    scratch_types=[
        pltpu.VMEM((N,), jnp.float32),
        pltpu.VMEM((N,), jnp.float32),
        pltpu.VMEM((PPT,), jnp.int32),
        pltpu.VMEM((PPT,), jnp.int32),
        pltpu.VMEM((PPT,), jnp.float32),
    ],
    compiler_params=pltpu.CompilerParams(needs_layout_passes=False),
)
def _sc_pairs(s_hbm, t_hbm, ei0_hbm, ei1_hbm, out_hbm,
              s_v, t_v, i0_v, i1_v, ob_v):
    w = _wid()
    base = pl.multiple_of(w * PPT, 8)
    pltpu.sync_copy(s_hbm, s_v)
    pltpu.sync_copy(t_hbm, t_v)
    pltpu.sync_copy(ei0_hbm.at[pl.ds(base, PPT)], i0_v)
    pltpu.sync_copy(ei1_hbm.at[pl.ds(base, PPT)], i1_v)

    def body(j, _):
        sl = pl.ds(j * 16, 16)
        v0 = plsc.load_gather(s_v, [i0_v[sl]])
        v1 = plsc.load_gather(t_v, [i1_v[sl]])
        z = v0 + v1
        ob_v[sl] = 1.0 / (1.0 + jnp.exp(-z))
        return 0

    lax.fori_loop(0, PCH, body, 0)
    pltpu.sync_copy(ob_v, out_hbm.at[pl.ds(base, PPT)])


BN = 2000


def _dinv_of(deg_ref):
    deg = deg_ref[0, :, 0:1] + deg_ref[1, :, 0:1] + 1.0
    return lax.rsqrt(deg)


_DEG_SPEC = pl.BlockSpec((NC, BN, 16), lambda i: (0, i, 0))


def _split_store(out_ref, g):
    out_ref[0] = g[:, :HW]
    out_ref[1] = g[:, HW:]


def _scale_mm_body(x_ref, w_ref, deg_ref, g_ref):
    xw = jnp.dot(x_ref[...], w_ref[...], preferred_element_type=jnp.float32)
    _split_store(g_ref, _dinv_of(deg_ref) * xw)


def _tc_scale_mm(x, W, deg_parts):
    return pl.pallas_call(
        _scale_mm_body,
        grid=(N // BN,),
        in_specs=[
            pl.BlockSpec((BN, D), lambda i: (i, 0)),
            pl.BlockSpec((D, H), lambda i: (0, 0)),
            _DEG_SPEC,
        ],
        out_specs=pl.BlockSpec((NC, BN, HW), lambda i: (0, i, 0)),
        out_shape=jax.ShapeDtypeStruct((NC, N, HW), jnp.float32),
    )(x, W, deg_parts)


def _layer_body(parts_ref, g_ref, deg_ref, b_ref, w_ref, out_ref):
    dinv = _dinv_of(deg_ref)
    tot = jnp.concatenate([parts_ref[0] + g_ref[0], parts_ref[1] + g_ref[1]],
                          axis=1)
    h = jnp.maximum(dinv * tot + b_ref[...], 0.0)
    hw = jnp.dot(h, w_ref[...], preferred_element_type=jnp.float32)
    _split_store(out_ref, dinv * hw)


def _tc_layer(parts, g, deg_parts, b, W):
    return pl.pallas_call(
        _layer_body,
        grid=(N // BN,),
        in_specs=[
            pl.BlockSpec((NC, BN, HW), lambda i: (0, i, 0)),
            pl.BlockSpec((NC, BN, HW), lambda i: (0, i, 0)),
            _DEG_SPEC,
            pl.BlockSpec((1, H), lambda i: (0, 0)),
            pl.BlockSpec((H, H), lambda i: (0, 0)),
        ],
        out_specs=pl.BlockSpec((NC, BN, HW), lambda i: (0, i, 0)),
        out_shape=jax.ShapeDtypeStruct((NC, N, HW), jnp.float32),
    )(parts, g, deg_parts, b, W)


def _final_body(parts_ref, g_ref, deg_ref, b_ref, wc0_ref, wc1_ref, bc_ref,
                h_ref, s_ref, t_ref):
    dinv = _dinv_of(deg_ref)
    tot = jnp.concatenate([parts_ref[0] + g_ref[0], parts_ref[1] + g_ref[1]],
                          axis=1)
    h = jnp.maximum(dinv * tot + b_ref[...], 0.0)
    h_ref[...] = h
    s_ref[...] = jnp.sum(h * wc0_ref[...], axis=1, keepdims=True) + bc_ref[0]
    t_ref[...] = jnp.sum(h * wc1_ref[...], axis=1, keepdims=True)


def _tc_final(parts, g, deg_parts, b, wc0, wc1, bc):
    return pl.pallas_call(
        _final_body,
        grid=(N // BN,),
        in_specs=[
            pl.BlockSpec((NC, BN, HW), lambda i: (0, i, 0)),
            pl.BlockSpec((NC, BN, HW), lambda i: (0, i, 0)),
            _DEG_SPEC,
            pl.BlockSpec((1, H), lambda i: (0, 0)),
            pl.BlockSpec((1, H), lambda i: (0, 0)),
            pl.BlockSpec((1, H), lambda i: (0, 0)),
            pl.BlockSpec(memory_space=pltpu.SMEM),
        ],
        out_specs=[
            pl.BlockSpec((BN, H), lambda i: (i, 0)),
            pl.BlockSpec((BN, 1), lambda i: (i, 0)),
            pl.BlockSpec((BN, 1), lambda i: (i, 0)),
        ],
        out_shape=[
            jax.ShapeDtypeStruct((N, H), jnp.float32),
            jax.ShapeDtypeStruct((N, 1), jnp.float32),
            jax.ShapeDtypeStruct((N, 1), jnp.float32),
        ],
    )(parts, g, deg_parts, b, wc0, wc1, bc)


def kernel(x, edge_index_ppi, edge_index, W1, b1, W2, b2, Wc, bc):
    src = edge_index_ppi[0].reshape(NW, NCHUNK, K)
    dst = edge_index_ppi[1].reshape(NW, NCHUNK, K)

    ones16 = jnp.ones((K, 16), jnp.float32)
    zeros16 = jnp.zeros((N, 16), jnp.float32)
    zerosHW = jnp.zeros((N, HW), jnp.float32)

    deg_parts = _sc_degree(dst, ones16, zeros16)

    g1 = _tc_scale_mm(x, W1, deg_parts)
    p1 = _sc_scatter(g1, src, dst, zerosHW)
    g2 = _tc_layer(p1, g1, deg_parts, b1.reshape(1, H), W2)
    p2 = _sc_scatter(g2, src, dst, zerosHW)

    wc0 = Wc[:H, 0].reshape(1, H)
    wc1 = Wc[H:, 0].reshape(1, H)
    h2, s_col, t_col = _tc_final(p2, g2, deg_parts, b2.reshape(1, H), wc0,
                                 wc1, bc)

    pad = ECP - EC
    ei0 = jnp.pad(edge_index[0], (0, pad))
    ei1 = jnp.pad(edge_index[1], (0, pad))
    probs = _sc_pairs(s_col.reshape(N), t_col.reshape(N), ei0, ei1)
    return (h2, probs[:EC].reshape(EC, 1))
